# Initial kernel scaffold; baseline (speedup 1.0000x reference)
import jax
import jax.numpy as jnp
from jax.experimental import pallas as pl


def kernel(batch_box_preds, batch_cls_preds):
    raise NotImplementedError("write your pallas kernel here")


# speedup vs baseline: 77.8686x; 77.8686x over previous
import functools

import jax
import jax.numpy as jnp
from jax.experimental import pallas as pl
from jax.experimental.pallas import tpu as pltpu

NMS_PRE = 4096
NMS_POST = 512
NMS_THRESH = 0.8

R = 160
C = 128
PN = R * C


def _nms_body(boxes_ref, cls_ref, rois_ref, scores_ref, labels_ref,
              ms_ref, x1_ref, x2_ref, y1_ref, y2_ref, area_ref,
              sc_ref, lb_ref):
    x = boxes_ref[0, 0]
    y = boxes_ref[0, 1]
    dx = boxes_ref[0, 3]
    dy = boxes_ref[0, 4]
    x1_ref[...] = x - dx * 0.5
    x2_ref[...] = x + dx * 0.5
    y1_ref[...] = y - dy * 0.5
    y2_ref[...] = y + dy * 0.5
    area_ref[...] = dx * dy

    c0 = cls_ref[0, 0]
    c1 = cls_ref[0, 1]
    c2 = cls_ref[0, 2]
    s = jnp.maximum(jnp.maximum(c0, c1), c2)
    lb = jnp.where(c2 > jnp.maximum(c0, c1), 2, jnp.where(c1 > c0, 1, 0))
    sc_ref[...] = s
    lb_ref[...] = lb.astype(jnp.int32)

    flat = (jax.lax.broadcasted_iota(jnp.int32, (R, C), 0) * C
            + jax.lax.broadcasted_iota(jnp.int32, (R, C), 1))

    bits = jax.lax.bitcast_convert_type(s, jnp.int32)

    def bs_val(_, carry):
        lo, hi = carry
        mid = lo + jax.lax.div(hi - lo, 2)
        cnt = jnp.sum(jnp.where(bits >= mid, 1, 0))
        cond = cnt >= NMS_PRE
        return jnp.where(cond, mid, lo), jnp.where(cond, hi, mid)

    t_bits, _ = jax.lax.fori_loop(
        0, 30, bs_val, (jnp.int32(0), jnp.int32(0x40000000)))

    n_above = jnp.sum(jnp.where(bits > t_bits, 1, 0))
    n_ties = NMS_PRE - n_above

    def bs_idx(_, carry):
        lo, hi = carry
        mid = lo + jax.lax.div(hi - lo, 2)
        cnt = jnp.sum(jnp.where((bits == t_bits) & (flat <= mid), 1, 0))
        cond = cnt >= n_ties
        return jnp.where(cond, lo, mid), jnp.where(cond, mid, hi)

    _, tie_idx = jax.lax.fori_loop(
        0, 16, bs_idx, (jnp.int32(-1), jnp.int32(PN - 1)))

    valid0 = (bits > t_bits) | ((bits == t_bits) & (flat <= tie_idx))
    ms_ref[...] = jnp.where(valid0, s, -1.0)

    rois_ref[...] = jnp.zeros_like(rois_ref)
    scores_ref[...] = jnp.zeros_like(scores_ref)
    labels_ref[...] = jnp.ones_like(labels_ref)

    lane = jax.lax.broadcasted_iota(jnp.int32, (1, C), 1)

    def pick_f32(ref4_row, jc):
        return jnp.sum(jnp.where(lane == jc, ref4_row, 0.0), axis=1,
                       keepdims=True)

    def body(i, carry):
        ms = ms_ref[...]
        m = jnp.max(ms)
        keep = m >= 0.0
        j = jnp.min(jnp.where(ms == m, flat, jnp.int32(2 ** 30)))
        jr = jax.lax.div(j, C)
        jc = jax.lax.rem(j, C)

        vals = [pick_f32(boxes_ref[0, f, pl.ds(jr, 1), :], jc)
                for f in range(7)]
        sval = pick_f32(sc_ref[pl.ds(jr, 1), :], jc)
        lrow = lb_ref[pl.ds(jr, 1), :]
        lval = jnp.sum(jnp.where(lane == jc, lrow, 0), axis=1, keepdims=True)

        xj, yj, dxj, dyj = vals[0], vals[1], vals[3], vals[4]
        x1j = xj - dxj * 0.5
        x2j = xj + dxj * 0.5
        y1j = yj - dyj * 0.5
        y2j = yj + dyj * 0.5
        aj = dxj * dyj

        iw = jnp.maximum(
            jnp.minimum(x2j, x2_ref[...]) - jnp.maximum(x1j, x1_ref[...]),
            0.0)
        ih = jnp.maximum(
            jnp.minimum(y2j, y2_ref[...]) - jnp.maximum(y1j, y1_ref[...]),
            0.0)
        inter = iw * ih
        supp = inter > NMS_THRESH * (aj + area_ref[...] - inter + 1e-8)
        ms_ref[...] = jnp.where(keep & (supp | (flat == j)), -1.0, ms)

        for f in range(7):
            rois_ref[0, pl.ds(i, 1), f:f + 1] = jnp.where(keep, vals[f], 0.0)
        scores_ref[0, pl.ds(i, 1), 0:1] = jnp.where(keep, sval, 0.0)
        labels_ref[0, pl.ds(i, 1), 0:1] = jnp.where(
            keep, lval + 1, jnp.int32(1))
        return carry

    jax.lax.fori_loop(0, NMS_POST, body, 0)


@jax.jit
def kernel(batch_box_preds, batch_cls_preds):
    B, N, _ = batch_box_preds.shape
    boxes = jnp.moveaxis(batch_box_preds, 2, 1)
    boxes = jnp.pad(boxes, ((0, 0), (0, 0), (0, PN - N)))
    boxes = boxes.reshape(B, 7, R, C)
    cls = jnp.moveaxis(batch_cls_preds, 2, 1)
    cls = jnp.pad(cls, ((0, 0), (0, 0), (0, PN - N)), constant_values=-1.0)
    cls = cls.reshape(B, 3, R, C)

    rois, scores, labels = pl.pallas_call(
        _nms_body,
        grid=(B,),
        in_specs=[
            pl.BlockSpec((1, 7, R, C), lambda b: (b, 0, 0, 0)),
            pl.BlockSpec((1, 3, R, C), lambda b: (b, 0, 0, 0)),
        ],
        out_specs=[
            pl.BlockSpec((1, NMS_POST, 7), lambda b: (b, 0, 0)),
            pl.BlockSpec((1, NMS_POST, 1), lambda b: (b, 0, 0)),
            pl.BlockSpec((1, NMS_POST, 1), lambda b: (b, 0, 0)),
        ],
        out_shape=[
            jax.ShapeDtypeStruct((B, NMS_POST, 7), jnp.float32),
            jax.ShapeDtypeStruct((B, NMS_POST, 1), jnp.float32),
            jax.ShapeDtypeStruct((B, NMS_POST, 1), jnp.int32),
        ],
        scratch_shapes=[pltpu.VMEM((R, C), jnp.float32)] * 7
        + [pltpu.VMEM((R, C), jnp.int32)],
    )(boxes, cls)
    return rois, scores.reshape(B, NMS_POST), labels.reshape(B, NMS_POST)


# speedup vs baseline: 125.2510x; 1.6085x over previous
import jax
import jax.numpy as jnp
from jax.experimental import pallas as pl
from jax.experimental.pallas import tpu as pltpu

NMS_PRE = 4096
NMS_POST = 512
NMS_THRESH = 0.8

R = 160
C = 128
PN = R * C


def _nms_body(boxes_ref, cls_ref, rois_ref, scores_ref, labels_ref,
              ms_ref, x1_ref, x2_ref, y1_ref, y2_ref, ta_ref,
              sc_ref, lb_ref):
    B = boxes_ref.shape[0]
    flat = (jax.lax.broadcasted_iota(jnp.int32, (R, C), 0) * C
            + jax.lax.broadcasted_iota(jnp.int32, (R, C), 1))
    lane = jax.lax.broadcasted_iota(jnp.int32, (1, C), 1)

    for b in range(B):
        x = boxes_ref[b, 0]
        y = boxes_ref[b, 1]
        dx = boxes_ref[b, 3]
        dy = boxes_ref[b, 4]
        x1_ref[b] = x - dx * 0.5
        x2_ref[b] = x + dx * 0.5
        y1_ref[b] = y - dy * 0.5
        y2_ref[b] = y + dy * 0.5
        ta_ref[b] = (dx * dy) * NMS_THRESH
        c0 = cls_ref[b, 0]
        c1 = cls_ref[b, 1]
        c2 = cls_ref[b, 2]
        sc_ref[b] = jnp.maximum(jnp.maximum(c0, c1), c2)
        lb_ref[b] = jnp.where(c2 > jnp.maximum(c0, c1), 2,
                              jnp.where(c1 > c0, 1, 0)).astype(jnp.int32)

    def bs_val(_, carry):
        lo, hi = carry
        nlo, nhi = [], []
        for b in range(B):
            bits = jax.lax.bitcast_convert_type(sc_ref[b], jnp.int32)
            mid = lo[b] + jax.lax.div(hi[b] - lo[b], 2)
            cnt = jnp.sum(jnp.where(bits >= mid, 1, 0))
            cond = cnt >= NMS_PRE
            nlo.append(jnp.where(cond, mid, lo[b]))
            nhi.append(jnp.where(cond, hi[b], mid))
        return tuple(nlo), tuple(nhi)

    t_bits, _ = jax.lax.fori_loop(
        0, 30, bs_val,
        ((jnp.int32(0),) * B, (jnp.int32(0x40000000),) * B))

    n_ties = []
    for b in range(B):
        bits = jax.lax.bitcast_convert_type(sc_ref[b], jnp.int32)
        n_ties.append(NMS_PRE - jnp.sum(jnp.where(bits > t_bits[b], 1, 0)))

    def bs_idx(_, carry):
        lo, hi = carry
        nlo, nhi = [], []
        for b in range(B):
            bits = jax.lax.bitcast_convert_type(sc_ref[b], jnp.int32)
            mid = lo[b] + jax.lax.div(hi[b] - lo[b], 2)
            cnt = jnp.sum(
                jnp.where((bits == t_bits[b]) & (flat <= mid), 1, 0))
            cond = cnt >= n_ties[b]
            nlo.append(jnp.where(cond, lo[b], mid))
            nhi.append(jnp.where(cond, mid, hi[b]))
        return tuple(nlo), tuple(nhi)

    _, tie_idx = jax.lax.fori_loop(
        0, 16, bs_idx,
        ((jnp.int32(-1),) * B, (jnp.int32(PN - 1),) * B))

    for b in range(B):
        bits = jax.lax.bitcast_convert_type(sc_ref[b], jnp.int32)
        valid0 = (bits > t_bits[b]) | ((bits == t_bits[b])
                                       & (flat <= tie_idx[b]))
        ms_ref[b] = jnp.where(valid0, sc_ref[b], -1.0)

    rois_ref[...] = jnp.zeros_like(rois_ref)
    scores_ref[...] = jnp.zeros_like(scores_ref)
    labels_ref[...] = jnp.ones_like(labels_ref)

    def pick_f32(row, jc):
        return jnp.sum(jnp.where(lane == jc, row, 0.0), axis=1,
                       keepdims=True)

    def body(i, carry):
        for b in range(B):
            ms = ms_ref[b]
            m = jnp.max(ms)
            keep = m >= 0.0
            j = jnp.min(jnp.where(ms == m, flat, jnp.int32(2 ** 30)))
            jr = jax.lax.div(j, C)
            jc = jax.lax.rem(j, C)

            vals = [pick_f32(boxes_ref[b, f, pl.ds(jr, 1), :], jc)
                    for f in range(7)]
            sval = pick_f32(sc_ref[b, pl.ds(jr, 1), :], jc)
            lrow = lb_ref[b, pl.ds(jr, 1), :]
            lval = jnp.sum(jnp.where(lane == jc, lrow, 0), axis=1,
                           keepdims=True)

            xj, yj, dxj, dyj = vals[0], vals[1], vals[3], vals[4]
            x1j = xj - dxj * 0.5
            x2j = xj + dxj * 0.5
            y1j = yj - dyj * 0.5
            y2j = yj + dyj * 0.5
            saj = NMS_THRESH * (dxj * dyj + 1e-8)

            iw = jnp.maximum(
                jnp.minimum(x2j, x2_ref[b]) - jnp.maximum(x1j, x1_ref[b]),
                0.0)
            ih = jnp.maximum(
                jnp.minimum(y2j, y2_ref[b]) - jnp.maximum(y1j, y1_ref[b]),
                0.0)
            inter = iw * ih
            supp = (1.0 + NMS_THRESH) * inter > saj + ta_ref[b]
            ms_ref[b] = jnp.where(keep & (supp | (flat == j)), -1.0, ms)

            for f in range(7):
                rois_ref[b, pl.ds(i, 1), f:f + 1] = jnp.where(
                    keep, vals[f], 0.0)
            scores_ref[b, pl.ds(i, 1), 0:1] = jnp.where(keep, sval, 0.0)
            labels_ref[b, pl.ds(i, 1), 0:1] = jnp.where(
                keep, lval + 1, jnp.int32(1))
        return carry

    jax.lax.fori_loop(0, NMS_POST, body, 0)


@jax.jit
def kernel(batch_box_preds, batch_cls_preds):
    B, N, _ = batch_box_preds.shape
    boxes = jnp.moveaxis(batch_box_preds, 2, 1)
    boxes = jnp.pad(boxes, ((0, 0), (0, 0), (0, PN - N)))
    boxes = boxes.reshape(B, 7, R, C)
    cls = jnp.moveaxis(batch_cls_preds, 2, 1)
    cls = jnp.pad(cls, ((0, 0), (0, 0), (0, PN - N)), constant_values=-1.0)
    cls = cls.reshape(B, 3, R, C)

    rois, scores, labels = pl.pallas_call(
        _nms_body,
        in_specs=[
            pl.BlockSpec((B, 7, R, C), lambda: (0, 0, 0, 0)),
            pl.BlockSpec((B, 3, R, C), lambda: (0, 0, 0, 0)),
        ],
        out_specs=[
            pl.BlockSpec((B, NMS_POST, 7), lambda: (0, 0, 0)),
            pl.BlockSpec((B, NMS_POST, 1), lambda: (0, 0, 0)),
            pl.BlockSpec((B, NMS_POST, 1), lambda: (0, 0, 0)),
        ],
        out_shape=[
            jax.ShapeDtypeStruct((B, NMS_POST, 7), jnp.float32),
            jax.ShapeDtypeStruct((B, NMS_POST, 1), jnp.float32),
            jax.ShapeDtypeStruct((B, NMS_POST, 1), jnp.int32),
        ],
        scratch_shapes=[pltpu.VMEM((B, R, C), jnp.float32)] * 7
        + [pltpu.VMEM((B, R, C), jnp.int32)],
    )(boxes, cls)
    return rois, scores.reshape(B, NMS_POST), labels.reshape(B, NMS_POST)


# speedup vs baseline: 129.0261x; 1.0301x over previous
import functools

import jax
import jax.numpy as jnp
from jax.experimental import pallas as pl
from jax.experimental.pallas import tpu as pltpu
from jax.experimental.pallas import tpu_sc as plsc

NMS_PRE = 4096
NMS_POST = 512
NMS_THRESH = 0.8

R = 160
C = 128
PN = R * C
NSLOT = 8
CH = PN // NSLOT
CW = CH + 32
OUTW = 8192
NR = NMS_PRE // C


def _thresh_body(cls_ref, sc_ref, lb_ref, thr_ref):
    B = cls_ref.shape[0]
    flat = (jax.lax.broadcasted_iota(jnp.int32, (R, C), 0) * C
            + jax.lax.broadcasted_iota(jnp.int32, (R, C), 1))

    for b in range(B):
        c0 = cls_ref[b, 0]
        c1 = cls_ref[b, 1]
        c2 = cls_ref[b, 2]
        sc_ref[b] = jnp.maximum(jnp.maximum(c0, c1), c2)
        lb_ref[b] = jnp.where(c2 > jnp.maximum(c0, c1), 2,
                              jnp.where(c1 > c0, 1, 0)).astype(jnp.int32)

    def bs_val(_, carry):
        lo, hi = carry
        nlo, nhi = [], []
        for b in range(B):
            bits = jax.lax.bitcast_convert_type(sc_ref[b], jnp.int32)
            mid = lo[b] + jax.lax.div(hi[b] - lo[b], 2)
            cnt = jnp.sum(jnp.where(bits >= mid, 1, 0))
            cond = cnt >= NMS_PRE
            nlo.append(jnp.where(cond, mid, lo[b]))
            nhi.append(jnp.where(cond, hi[b], mid))
        return tuple(nlo), tuple(nhi)

    t_bits, _ = jax.lax.fori_loop(
        0, 30, bs_val,
        ((jnp.int32(0),) * B, (jnp.int32(0x40000000),) * B))

    n_ties = []
    for b in range(B):
        bits = jax.lax.bitcast_convert_type(sc_ref[b], jnp.int32)
        n_ties.append(NMS_PRE - jnp.sum(jnp.where(bits > t_bits[b], 1, 0)))

    def bs_idx(_, carry):
        lo, hi = carry
        nlo, nhi = [], []
        for b in range(B):
            bits = jax.lax.bitcast_convert_type(sc_ref[b], jnp.int32)
            mid = lo[b] + jax.lax.div(hi[b] - lo[b], 2)
            cnt = jnp.sum(
                jnp.where((bits == t_bits[b]) & (flat <= mid), 1, 0))
            cond = cnt >= n_ties[b]
            nlo.append(jnp.where(cond, lo[b], mid))
            nhi.append(jnp.where(cond, mid, hi[b]))
        return tuple(nlo), tuple(nhi)

    _, tie_idx = jax.lax.fori_loop(
        0, 16, bs_idx,
        ((jnp.int32(-1),) * B, (jnp.int32(PN - 1),) * B))

    thr_ref[...] = jnp.zeros_like(thr_ref)
    for b in range(B):
        thr_ref[NSLOT * b:NSLOT * b + 1, 0:1] = jnp.full(
            (1, 1), t_bits[b], jnp.int32)
        thr_ref[NSLOT * b:NSLOT * b + 1, 1:2] = jnp.full(
            (1, 1), tie_idx[b], jnp.int32)


def _compact_body(boxflat, scflat, lbflat, thr, outf, outl,
                  bufin, lbin, bufout, lbout, thrv, cntv, cnts,
                  mrgf, mrgl, shared):
    c = jax.lax.axis_index("c")
    s = jax.lax.axis_index("s")
    batch = c * 2 + jax.lax.div(s, NSLOT)
    slot = jax.lax.rem(s, NSLOT)
    start = slot * CH
    lane16 = jax.lax.broadcasted_iota(jnp.int32, (16,), 0)

    pltpu.sync_copy(
        thr.at[pl.ds(pl.multiple_of(batch * NSLOT * C, 16), 16)], thrv)
    tv = thrv[...]
    t_bits = jnp.sum(jnp.where(lane16 == 0, tv, 0))
    tie = jnp.sum(jnp.where(lane16 == 1, tv, 0))

    for f in range(7):
        pltpu.sync_copy(
            boxflat.at[pl.ds(
                pl.multiple_of((batch * 7 + f) * PN + start, 16), CH)],
            bufin.at[pl.ds(f * CH, CH)])
    pltpu.sync_copy(
        scflat.at[pl.ds(pl.multiple_of(batch * PN + start, 16), CH)],
        bufin.at[pl.ds(7 * CH, CH)])
    pltpu.sync_copy(
        lbflat.at[pl.ds(pl.multiple_of(batch * PN + start, 16), CH)], lbin)

    def valid_at(g):
        sv = bufin[pl.ds(7 * CH + g * 16, 16)]
        bits = plsc.bitcast(sv, jnp.int32)
        gi = start + g * 16 + lane16
        return (bits > t_bits) | ((bits == t_bits) & (gi <= tie))

    def count_body(g, cnt):
        pc = plsc.all_reduce_population_count(valid_at(g))
        return cnt + jnp.sum(jnp.where(lane16 == 0, pc, 0))

    cnt = jax.lax.fori_loop(0, CH // 16, count_body, jnp.int32(0))

    cntv[...] = jnp.full((16,), cnt, jnp.int32)
    pltpu.sync_copy(cntv, shared.at[pl.ds(pl.multiple_of(s * 16, 16), 16)])
    plsc.subcore_barrier()
    base = jax.lax.div(s, NSLOT) * NSLOT
    pltpu.sync_copy(
        shared.at[pl.ds(pl.multiple_of(base * 16, 16), NSLOT * 16)], cnts)
    prefix = jnp.int32(0)
    for k in range(NSLOT):
        ck = jnp.sum(jnp.where(lane16 == 0, cnts[pl.ds(k * 16, 16)], 0))
        prefix = prefix + jnp.where(jnp.int32(k) < slot, ck, jnp.int32(0))
    astart = pl.multiple_of(prefix - jax.lax.rem(prefix, 16), 16)
    pad = prefix - astart

    def comp_body(g, off):
        valid = valid_at(g)
        for f in range(8):
            v = bufin[pl.ds(f * CH + g * 16, 16)]
            plsc.store_compressed(bufout.at[pl.ds(f * CW + off, 16)], v,
                                  mask=valid)
        lv = lbin[pl.ds(g * 16, 16)]
        plsc.store_compressed(lbout.at[pl.ds(off, 16)], lv, mask=valid)
        pc = plsc.all_reduce_population_count(valid)
        return off + jnp.sum(jnp.where(lane16 == 0, pc, 0))

    jax.lax.fori_loop(0, CH // 16, comp_body, pad)

    for r in range(NSLOT):
        plsc.subcore_barrier()

        @pl.when(slot == r)
        def _commit():
            for f in range(8):
                dst = pl.multiple_of((batch * 8 + f) * OUTW + astart, 16)
                pltpu.sync_copy(outf.at[pl.ds(dst, 16)], mrgf)
                head = bufout[pl.ds(f * CW, 16)]
                bufout[pl.ds(f * CW, 16)] = jnp.where(
                    lane16 < pad, mrgf[...], head)
                pltpu.sync_copy(bufout.at[pl.ds(f * CW, CW)],
                                outf.at[pl.ds(dst, CW)])
            ldst = pl.multiple_of(batch * OUTW + astart, 16)
            pltpu.sync_copy(outl.at[pl.ds(ldst, 16)], mrgl)
            lhead = lbout[pl.ds(0, 16)]
            lbout[pl.ds(0, 16)] = jnp.where(lane16 < pad, mrgl[...], lhead)
            pltpu.sync_copy(lbout, outl.at[pl.ds(ldst, CW)])


def _nms_body(boxc_ref, lbc_ref, rois_ref, scores_ref, labels_ref,
              ms_ref, x1_ref, x2_ref, y1_ref, y2_ref, ta_ref):
    B = boxc_ref.shape[0]
    flat = (jax.lax.broadcasted_iota(jnp.int32, (NR, C), 0) * C
            + jax.lax.broadcasted_iota(jnp.int32, (NR, C), 1))
    lane = jax.lax.broadcasted_iota(jnp.int32, (1, C), 1)

    for b in range(B):
        dx = boxc_ref[b, 3]
        dy = boxc_ref[b, 4]
        x1_ref[b] = boxc_ref[b, 0] - dx * 0.5
        x2_ref[b] = boxc_ref[b, 0] + dx * 0.5
        y1_ref[b] = boxc_ref[b, 1] - dy * 0.5
        y2_ref[b] = boxc_ref[b, 1] + dy * 0.5
        ta_ref[b] = (dx * dy) * NMS_THRESH
        ms_ref[b] = boxc_ref[b, 7]

    rois_ref[...] = jnp.zeros_like(rois_ref)
    scores_ref[...] = jnp.zeros_like(scores_ref)
    labels_ref[...] = jnp.ones_like(labels_ref)

    def pick_f32(row, jc):
        return jnp.sum(jnp.where(lane == jc, row, 0.0), axis=1,
                       keepdims=True)

    def body(i, carry):
        for b in range(B):
            ms = ms_ref[b]
            m = jnp.max(ms)
            keep = m >= 0.0
            j = jnp.min(jnp.where(ms == m, flat, jnp.int32(2 ** 30)))
            jr = jax.lax.div(j, C)
            jc = jax.lax.rem(j, C)

            vals = [pick_f32(boxc_ref[b, f, pl.ds(jr, 1), :], jc)
                    for f in range(8)]
            lrow = lbc_ref[b, pl.ds(jr, 1), :]
            lval = jnp.sum(jnp.where(lane == jc, lrow, 0), axis=1,
                           keepdims=True)

            xj, yj, dxj, dyj = vals[0], vals[1], vals[3], vals[4]
            x1j = xj - dxj * 0.5
            x2j = xj + dxj * 0.5
            y1j = yj - dyj * 0.5
            y2j = yj + dyj * 0.5
            saj = NMS_THRESH * (dxj * dyj + 1e-8)

            iw = jnp.maximum(
                jnp.minimum(x2j, x2_ref[b]) - jnp.maximum(x1j, x1_ref[b]),
                0.0)
            ih = jnp.maximum(
                jnp.minimum(y2j, y2_ref[b]) - jnp.maximum(y1j, y1_ref[b]),
                0.0)
            inter = iw * ih
            supp = (1.0 + NMS_THRESH) * inter > saj + ta_ref[b]
            ms_ref[b] = jnp.where(keep & (supp | (flat == j)), -1.0, ms)

            for f in range(7):
                rois_ref[b, pl.ds(i, 1), f:f + 1] = jnp.where(
                    keep, vals[f], 0.0)
            scores_ref[b, pl.ds(i, 1), 0:1] = jnp.where(keep, vals[7], 0.0)
            labels_ref[b, pl.ds(i, 1), 0:1] = jnp.where(
                keep, lval + 1, jnp.int32(1))
        return carry

    jax.lax.fori_loop(0, NMS_POST, body, 0)


@jax.jit
def kernel(batch_box_preds, batch_cls_preds):
    B, N, _ = batch_box_preds.shape
    boxes = jnp.moveaxis(batch_box_preds, 2, 1)
    boxes = jnp.pad(boxes, ((0, 0), (0, 0), (0, PN - N)))
    boxes = boxes.reshape(B, 7, R, C)
    cls = jnp.moveaxis(batch_cls_preds, 2, 1)
    cls = jnp.pad(cls, ((0, 0), (0, 0), (0, PN - N)), constant_values=-1.0)
    cls = cls.reshape(B, 3, R, C)

    scp, lbp, thr = pl.pallas_call(
        _thresh_body,
        in_specs=[
            pl.BlockSpec((B, 3, R, C), lambda: (0, 0, 0, 0)),
        ],
        out_specs=[
            pl.BlockSpec((B, R, C), lambda: (0, 0, 0)),
            pl.BlockSpec((B, R, C), lambda: (0, 0, 0)),
            pl.BlockSpec((B * NSLOT, C), lambda: (0, 0)),
        ],
        out_shape=[
            jax.ShapeDtypeStruct((B, R, C), jnp.float32),
            jax.ShapeDtypeStruct((B, R, C), jnp.int32),
            jax.ShapeDtypeStruct((B * NSLOT, C), jnp.int32),
        ],
    )(cls)

    compact = pl.kernel(
        _compact_body,
        out_type=[
            jax.ShapeDtypeStruct((B * 8 * OUTW,), jnp.float32),
            jax.ShapeDtypeStruct((B * OUTW,), jnp.int32),
        ],
        mesh=plsc.VectorSubcoreMesh(core_axis_name="c",
                                    subcore_axis_name="s",
                                    num_cores=2, num_subcores=16),
        compiler_params=pltpu.CompilerParams(needs_layout_passes=False),
        scratch_types=[
            pltpu.VMEM((8 * CH,), jnp.float32),
            pltpu.VMEM((CH,), jnp.int32),
            pltpu.VMEM((8 * CW,), jnp.float32),
            pltpu.VMEM((CW,), jnp.int32),
            pltpu.VMEM((16,), jnp.int32),
            pltpu.VMEM((16,), jnp.int32),
            pltpu.VMEM((NSLOT * 16,), jnp.int32),
            pltpu.VMEM((16,), jnp.float32),
            pltpu.VMEM((16,), jnp.int32),
            pltpu.VMEM_SHARED((16 * 16,), jnp.int32),
        ],
    )
    outf, outl = compact(boxes.reshape(B * 7 * PN), scp.reshape(B * PN),
                         lbp.reshape(B * PN), thr.reshape(B * NSLOT * C))

    boxc = outf.reshape(B, 8, OUTW // C, C)
    lbc = outl.reshape(B, OUTW // C, C)

    rois, scores, labels = pl.pallas_call(
        _nms_body,
        grid=(1,),
        in_specs=[
            pl.BlockSpec((B, 8, NR, C), lambda i: (0, 0, 0, 0)),
            pl.BlockSpec((B, NR, C), lambda i: (0, 0, 0)),
        ],
        out_specs=[
            pl.BlockSpec((B, NMS_POST, 7), lambda i: (0, 0, 0)),
            pl.BlockSpec((B, NMS_POST, 1), lambda i: (0, 0, 0)),
            pl.BlockSpec((B, NMS_POST, 1), lambda i: (0, 0, 0)),
        ],
        out_shape=[
            jax.ShapeDtypeStruct((B, NMS_POST, 7), jnp.float32),
            jax.ShapeDtypeStruct((B, NMS_POST, 1), jnp.float32),
            jax.ShapeDtypeStruct((B, NMS_POST, 1), jnp.int32),
        ],
        scratch_shapes=[pltpu.VMEM((B, NR, C), jnp.float32)] * 6,
    )(boxc, lbc)
    return rois, scores.reshape(B, NMS_POST), labels.reshape(B, NMS_POST)


# speedup vs baseline: 221.5372x; 1.7170x over previous
import functools

import jax
import jax.numpy as jnp
from jax.experimental import pallas as pl
from jax.experimental.pallas import tpu as pltpu
from jax.experimental.pallas import tpu_sc as plsc

NMS_PRE = 4096
NMS_POST = 512
NMS_THRESH = 0.8

R = 160
C = 128
PN = R * C
NSLOT = 8
CH = PN // NSLOT
CW = CH + 32
OUTW = 8192
NR = NMS_PRE // C


def _thresh_body(cls_ref, sc_ref, lb_ref, thr_ref):
    B = cls_ref.shape[0]
    flat = (jax.lax.broadcasted_iota(jnp.int32, (R, C), 0) * C
            + jax.lax.broadcasted_iota(jnp.int32, (R, C), 1))

    for b in range(B):
        c0 = cls_ref[b, 0]
        c1 = cls_ref[b, 1]
        c2 = cls_ref[b, 2]
        sc_ref[b] = jnp.maximum(jnp.maximum(c0, c1), c2)
        lb_ref[b] = jnp.where(c2 > jnp.maximum(c0, c1), 2,
                              jnp.where(c1 > c0, 1, 0)).astype(jnp.int32)

    def bs_val(_, carry):
        lo, hi = carry
        nlo, nhi = [], []
        for b in range(B):
            bits = jax.lax.bitcast_convert_type(sc_ref[b], jnp.int32)
            mid = lo[b] + jax.lax.div(hi[b] - lo[b], 2)
            cnt = jnp.sum(jnp.where(bits >= mid, 1, 0))
            cond = cnt >= NMS_PRE
            nlo.append(jnp.where(cond, mid, lo[b]))
            nhi.append(jnp.where(cond, hi[b], mid))
        return tuple(nlo), tuple(nhi)

    t_bits, _ = jax.lax.fori_loop(
        0, 30, bs_val,
        ((jnp.int32(0),) * B, (jnp.int32(0x40000000),) * B))

    n_ties = []
    for b in range(B):
        bits = jax.lax.bitcast_convert_type(sc_ref[b], jnp.int32)
        n_ties.append(NMS_PRE - jnp.sum(jnp.where(bits > t_bits[b], 1, 0)))

    def bs_idx(_, carry):
        lo, hi = carry
        nlo, nhi = [], []
        for b in range(B):
            bits = jax.lax.bitcast_convert_type(sc_ref[b], jnp.int32)
            mid = lo[b] + jax.lax.div(hi[b] - lo[b], 2)
            cnt = jnp.sum(
                jnp.where((bits == t_bits[b]) & (flat <= mid), 1, 0))
            cond = cnt >= n_ties[b]
            nlo.append(jnp.where(cond, lo[b], mid))
            nhi.append(jnp.where(cond, mid, hi[b]))
        return tuple(nlo), tuple(nhi)

    _, tie_idx = jax.lax.fori_loop(
        0, 16, bs_idx,
        ((jnp.int32(-1),) * B, (jnp.int32(PN - 1),) * B))

    thr_ref[...] = jnp.zeros_like(thr_ref)
    for b in range(B):
        thr_ref[NSLOT * b:NSLOT * b + 1, 0:1] = jnp.full(
            (1, 1), t_bits[b], jnp.int32)
        thr_ref[NSLOT * b:NSLOT * b + 1, 1:2] = jnp.full(
            (1, 1), tie_idx[b], jnp.int32)


def _compact_body(boxflat, scflat, lbflat, thr, outf, outl,
                  bufin, lbin, bufout, lbout, thrv, cntv, cnts,
                  mrgf, mrgl, shared):
    c = jax.lax.axis_index("c")
    s = jax.lax.axis_index("s")
    batch = c * 2 + jax.lax.div(s, NSLOT)
    slot = jax.lax.rem(s, NSLOT)
    start = slot * CH
    lane16 = jax.lax.broadcasted_iota(jnp.int32, (16,), 0)

    pltpu.sync_copy(
        thr.at[pl.ds(pl.multiple_of(batch * NSLOT * C, 16), 16)], thrv)
    tv = thrv[...]
    t_bits = jnp.sum(jnp.where(lane16 == 0, tv, 0))
    tie = jnp.sum(jnp.where(lane16 == 1, tv, 0))

    for f in range(7):
        pltpu.sync_copy(
            boxflat.at[pl.ds(
                pl.multiple_of((batch * 7 + f) * PN + start, 16), CH)],
            bufin.at[pl.ds(f * CH, CH)])
    pltpu.sync_copy(
        scflat.at[pl.ds(pl.multiple_of(batch * PN + start, 16), CH)],
        bufin.at[pl.ds(7 * CH, CH)])
    pltpu.sync_copy(
        lbflat.at[pl.ds(pl.multiple_of(batch * PN + start, 16), CH)], lbin)

    def valid_at(g):
        sv = bufin[pl.ds(7 * CH + g * 16, 16)]
        bits = plsc.bitcast(sv, jnp.int32)
        gi = start + g * 16 + lane16
        return (bits > t_bits) | ((bits == t_bits) & (gi <= tie))

    def count_body(g, cnt):
        pc = plsc.all_reduce_population_count(valid_at(g))
        return cnt + jnp.sum(jnp.where(lane16 == 0, pc, 0))

    cnt = jax.lax.fori_loop(0, CH // 16, count_body, jnp.int32(0))

    cntv[...] = jnp.full((16,), cnt, jnp.int32)
    pltpu.sync_copy(cntv, shared.at[pl.ds(pl.multiple_of(s * 16, 16), 16)])
    plsc.subcore_barrier()
    base = jax.lax.div(s, NSLOT) * NSLOT
    pltpu.sync_copy(
        shared.at[pl.ds(pl.multiple_of(base * 16, 16), NSLOT * 16)], cnts)
    prefix = jnp.int32(0)
    for k in range(NSLOT):
        ck = jnp.sum(jnp.where(lane16 == 0, cnts[pl.ds(k * 16, 16)], 0))
        prefix = prefix + jnp.where(jnp.int32(k) < slot, ck, jnp.int32(0))
    astart = pl.multiple_of(prefix - jax.lax.rem(prefix, 16), 16)
    pad = prefix - astart

    def comp_body(g, off):
        valid = valid_at(g)
        for f in range(8):
            v = bufin[pl.ds(f * CH + g * 16, 16)]
            plsc.store_compressed(bufout.at[pl.ds(f * CW + off, 16)], v,
                                  mask=valid)
        lv = lbin[pl.ds(g * 16, 16)]
        plsc.store_compressed(lbout.at[pl.ds(off, 16)], lv, mask=valid)
        pc = plsc.all_reduce_population_count(valid)
        return off + jnp.sum(jnp.where(lane16 == 0, pc, 0))

    jax.lax.fori_loop(0, CH // 16, comp_body, pad)

    for r in range(NSLOT):
        plsc.subcore_barrier()

        @pl.when(slot == r)
        def _commit():
            for f in range(8):
                dst = pl.multiple_of((batch * 8 + f) * OUTW + astart, 16)
                pltpu.sync_copy(outf.at[pl.ds(dst, 16)], mrgf)
                head = bufout[pl.ds(f * CW, 16)]
                bufout[pl.ds(f * CW, 16)] = jnp.where(
                    lane16 < pad, mrgf[...], head)
                pltpu.sync_copy(bufout.at[pl.ds(f * CW, CW)],
                                outf.at[pl.ds(dst, CW)])
            ldst = pl.multiple_of(batch * OUTW + astart, 16)
            pltpu.sync_copy(outl.at[pl.ds(ldst, 16)], mrgl)
            lhead = lbout[pl.ds(0, 16)]
            lbout[pl.ds(0, 16)] = jnp.where(lane16 < pad, mrgl[...], lhead)
            pltpu.sync_copy(lbout, outl.at[pl.ds(ldst, CW)])


def _nms_body(boxc_ref, lbc_ref, rois_ref, scores_ref, labels_ref,
              x1_ref, x2_ref, y1_ref, y2_ref, ta_ref):
    B = boxc_ref.shape[0]
    flat = (jax.lax.broadcasted_iota(jnp.int32, (NR, C), 0) * C
            + jax.lax.broadcasted_iota(jnp.int32, (NR, C), 1))
    lane = jax.lax.broadcasted_iota(jnp.int32, (1, C), 1)

    for b in range(B):
        dx = boxc_ref[b, 3]
        dy = boxc_ref[b, 4]
        x1_ref[b] = boxc_ref[b, 0] - dx * 0.5
        x2_ref[b] = boxc_ref[b, 0] + dx * 0.5
        y1_ref[b] = boxc_ref[b, 1] - dy * 0.5
        y2_ref[b] = boxc_ref[b, 1] + dy * 0.5
        ta_ref[b] = (dx * dy) * NMS_THRESH

    rois_ref[...] = jnp.zeros_like(rois_ref)
    scores_ref[...] = jnp.zeros_like(scores_ref)
    labels_ref[...] = jnp.ones_like(labels_ref)

    def red_max(a):
        return jnp.max(jnp.max(a, axis=1, keepdims=True), axis=0,
                       keepdims=True)

    def red_min(a):
        return jnp.min(jnp.min(a, axis=1, keepdims=True), axis=0,
                       keepdims=True)

    def red_sum(a):
        return jnp.sum(jnp.sum(a, axis=1, keepdims=True), axis=0,
                       keepdims=True)

    def body(i, mss):
        out = []
        for b in range(B):
            ms = mss[b]
            m = red_max(ms)
            keep = m >= 0.0
            jv = red_min(jnp.where(ms == m, flat, jnp.int32(2 ** 30)))
            onehot = flat == jv

            vals = [red_sum(jnp.where(onehot, boxc_ref[b, f], 0.0))
                    for f in range(8)]
            lval = red_sum(jnp.where(onehot, lbc_ref[b], 0))

            xj, yj, dxj, dyj = vals[0], vals[1], vals[3], vals[4]
            x1j = xj - dxj * 0.5
            x2j = xj + dxj * 0.5
            y1j = yj - dyj * 0.5
            y2j = yj + dyj * 0.5
            saj = NMS_THRESH * (dxj * dyj + 1e-8)

            iw = jnp.maximum(
                jnp.minimum(x2j, x2_ref[b]) - jnp.maximum(x1j, x1_ref[b]),
                0.0)
            ih = jnp.maximum(
                jnp.minimum(y2j, y2_ref[b]) - jnp.maximum(y1j, y1_ref[b]),
                0.0)
            inter = iw * ih
            supp = (1.0 + NMS_THRESH) * inter > saj + ta_ref[b]
            out.append(jnp.where(keep & (supp | onehot), -1.0, ms))

            for f in range(7):
                rois_ref[b, pl.ds(i, 1), f:f + 1] = jnp.where(
                    keep, vals[f], 0.0)
            scores_ref[b, pl.ds(i, 1), 0:1] = jnp.where(keep, vals[7], 0.0)
            labels_ref[b, pl.ds(i, 1), 0:1] = jnp.where(
                keep, lval + 1, jnp.int32(1))
        return tuple(out)

    jax.lax.fori_loop(0, NMS_POST, body,
                      tuple(boxc_ref[b, 7] for b in range(B)))


@jax.jit
def kernel(batch_box_preds, batch_cls_preds):
    B, N, _ = batch_box_preds.shape
    boxes = jnp.moveaxis(batch_box_preds, 2, 1)
    boxes = jnp.pad(boxes, ((0, 0), (0, 0), (0, PN - N)))
    boxes = boxes.reshape(B, 7, R, C)
    cls = jnp.moveaxis(batch_cls_preds, 2, 1)
    cls = jnp.pad(cls, ((0, 0), (0, 0), (0, PN - N)), constant_values=-1.0)
    cls = cls.reshape(B, 3, R, C)

    scp, lbp, thr = pl.pallas_call(
        _thresh_body,
        in_specs=[
            pl.BlockSpec((B, 3, R, C), lambda: (0, 0, 0, 0)),
        ],
        out_specs=[
            pl.BlockSpec((B, R, C), lambda: (0, 0, 0)),
            pl.BlockSpec((B, R, C), lambda: (0, 0, 0)),
            pl.BlockSpec((B * NSLOT, C), lambda: (0, 0)),
        ],
        out_shape=[
            jax.ShapeDtypeStruct((B, R, C), jnp.float32),
            jax.ShapeDtypeStruct((B, R, C), jnp.int32),
            jax.ShapeDtypeStruct((B * NSLOT, C), jnp.int32),
        ],
    )(cls)

    compact = pl.kernel(
        _compact_body,
        out_type=[
            jax.ShapeDtypeStruct((B * 8 * OUTW,), jnp.float32),
            jax.ShapeDtypeStruct((B * OUTW,), jnp.int32),
        ],
        mesh=plsc.VectorSubcoreMesh(core_axis_name="c",
                                    subcore_axis_name="s",
                                    num_cores=2, num_subcores=16),
        compiler_params=pltpu.CompilerParams(needs_layout_passes=False),
        scratch_types=[
            pltpu.VMEM((8 * CH,), jnp.float32),
            pltpu.VMEM((CH,), jnp.int32),
            pltpu.VMEM((8 * CW,), jnp.float32),
            pltpu.VMEM((CW,), jnp.int32),
            pltpu.VMEM((16,), jnp.int32),
            pltpu.VMEM((16,), jnp.int32),
            pltpu.VMEM((NSLOT * 16,), jnp.int32),
            pltpu.VMEM((16,), jnp.float32),
            pltpu.VMEM((16,), jnp.int32),
            pltpu.VMEM_SHARED((16 * 16,), jnp.int32),
        ],
    )
    outf, outl = compact(boxes.reshape(B * 7 * PN), scp.reshape(B * PN),
                         lbp.reshape(B * PN), thr.reshape(B * NSLOT * C))

    boxc = outf.reshape(B, 8, OUTW // C, C)
    lbc = outl.reshape(B, OUTW // C, C)

    rois, scores, labels = pl.pallas_call(
        _nms_body,
        grid=(1,),
        in_specs=[
            pl.BlockSpec((B, 8, NR, C), lambda i: (0, 0, 0, 0)),
            pl.BlockSpec((B, NR, C), lambda i: (0, 0, 0)),
        ],
        out_specs=[
            pl.BlockSpec((B, NMS_POST, 7), lambda i: (0, 0, 0)),
            pl.BlockSpec((B, NMS_POST, 1), lambda i: (0, 0, 0)),
            pl.BlockSpec((B, NMS_POST, 1), lambda i: (0, 0, 0)),
        ],
        out_shape=[
            jax.ShapeDtypeStruct((B, NMS_POST, 7), jnp.float32),
            jax.ShapeDtypeStruct((B, NMS_POST, 1), jnp.float32),
            jax.ShapeDtypeStruct((B, NMS_POST, 1), jnp.int32),
        ],
        scratch_shapes=[pltpu.VMEM((B, NR, C), jnp.float32)] * 5,
    )(boxc, lbc)
    return rois, scores.reshape(B, NMS_POST), labels.reshape(B, NMS_POST)


# speedup vs baseline: 238.9832x; 1.0787x over previous
import functools

import jax
import jax.numpy as jnp
from jax.experimental import pallas as pl
from jax.experimental.pallas import tpu as pltpu
from jax.experimental.pallas import tpu_sc as plsc

NMS_PRE = 4096
NMS_POST = 512
NMS_THRESH = 0.8

R = 160
C = 128
PN = R * C
NSLOT = 8
CH = PN // NSLOT
CW = CH + 32
OUTW = 8192
NR = NMS_PRE // C


def _thresh_body(cls_ref, sc_ref, lb_ref, thr_ref):
    B = cls_ref.shape[0]
    flat = (jax.lax.broadcasted_iota(jnp.int32, (R, C), 0) * C
            + jax.lax.broadcasted_iota(jnp.int32, (R, C), 1))

    for b in range(B):
        c0 = cls_ref[b, 0]
        c1 = cls_ref[b, 1]
        c2 = cls_ref[b, 2]
        sc_ref[b] = jnp.maximum(jnp.maximum(c0, c1), c2)
        lb_ref[b] = jnp.where(c2 > jnp.maximum(c0, c1), 2,
                              jnp.where(c1 > c0, 1, 0)).astype(jnp.int32)

    def bs_val(_, carry):
        lo, hi = carry
        nlo, nhi = [], []
        for b in range(B):
            bits = jax.lax.bitcast_convert_type(sc_ref[b], jnp.int32)
            mid = lo[b] + jax.lax.div(hi[b] - lo[b], 2)
            cnt = jnp.sum(jnp.where(bits >= mid, 1, 0))
            cond = cnt >= NMS_PRE
            nlo.append(jnp.where(cond, mid, lo[b]))
            nhi.append(jnp.where(cond, hi[b], mid))
        return tuple(nlo), tuple(nhi)

    t_bits, _ = jax.lax.fori_loop(
        0, 30, bs_val,
        ((jnp.int32(0),) * B, (jnp.int32(0x40000000),) * B))

    n_ties = []
    for b in range(B):
        bits = jax.lax.bitcast_convert_type(sc_ref[b], jnp.int32)
        n_ties.append(NMS_PRE - jnp.sum(jnp.where(bits > t_bits[b], 1, 0)))

    def bs_idx(_, carry):
        lo, hi = carry
        nlo, nhi = [], []
        for b in range(B):
            bits = jax.lax.bitcast_convert_type(sc_ref[b], jnp.int32)
            mid = lo[b] + jax.lax.div(hi[b] - lo[b], 2)
            cnt = jnp.sum(
                jnp.where((bits == t_bits[b]) & (flat <= mid), 1, 0))
            cond = cnt >= n_ties[b]
            nlo.append(jnp.where(cond, lo[b], mid))
            nhi.append(jnp.where(cond, mid, hi[b]))
        return tuple(nlo), tuple(nhi)

    _, tie_idx = jax.lax.fori_loop(
        0, 16, bs_idx,
        ((jnp.int32(-1),) * B, (jnp.int32(PN - 1),) * B))

    thr_ref[...] = jnp.zeros_like(thr_ref)
    for b in range(B):
        thr_ref[NSLOT * b:NSLOT * b + 1, 0:1] = jnp.full(
            (1, 1), t_bits[b], jnp.int32)
        thr_ref[NSLOT * b:NSLOT * b + 1, 1:2] = jnp.full(
            (1, 1), tie_idx[b], jnp.int32)


def _compact_body(boxflat, scflat, lbflat, thr, outf, outl,
                  bufin, lbin, bufout, lbout, thrv, cntv, cnts,
                  mrgf, mrgl, shared):
    c = jax.lax.axis_index("c")
    s = jax.lax.axis_index("s")
    batch = c * 2 + jax.lax.div(s, NSLOT)
    slot = jax.lax.rem(s, NSLOT)
    start = slot * CH
    lane16 = jax.lax.broadcasted_iota(jnp.int32, (16,), 0)

    pltpu.sync_copy(
        thr.at[pl.ds(pl.multiple_of(batch * NSLOT * C, 16), 16)], thrv)
    tv = thrv[...]
    t_bits = jnp.sum(jnp.where(lane16 == 0, tv, 0))
    tie = jnp.sum(jnp.where(lane16 == 1, tv, 0))

    for f in range(7):
        pltpu.sync_copy(
            boxflat.at[pl.ds(
                pl.multiple_of((batch * 7 + f) * PN + start, 16), CH)],
            bufin.at[pl.ds(f * CH, CH)])
    pltpu.sync_copy(
        scflat.at[pl.ds(pl.multiple_of(batch * PN + start, 16), CH)],
        bufin.at[pl.ds(7 * CH, CH)])
    pltpu.sync_copy(
        lbflat.at[pl.ds(pl.multiple_of(batch * PN + start, 16), CH)], lbin)

    def valid_at(g):
        sv = bufin[pl.ds(7 * CH + g * 16, 16)]
        bits = plsc.bitcast(sv, jnp.int32)
        gi = start + g * 16 + lane16
        return (bits > t_bits) | ((bits == t_bits) & (gi <= tie))

    def count_body(g, cnt):
        pc = plsc.all_reduce_population_count(valid_at(g))
        return cnt + jnp.sum(jnp.where(lane16 == 0, pc, 0))

    cnt = jax.lax.fori_loop(0, CH // 16, count_body, jnp.int32(0))

    cntv[...] = jnp.full((16,), cnt, jnp.int32)
    pltpu.sync_copy(cntv, shared.at[pl.ds(pl.multiple_of(s * 16, 16), 16)])
    plsc.subcore_barrier()
    base = jax.lax.div(s, NSLOT) * NSLOT
    pltpu.sync_copy(
        shared.at[pl.ds(pl.multiple_of(base * 16, 16), NSLOT * 16)], cnts)
    prefix = jnp.int32(0)
    for k in range(NSLOT):
        ck = jnp.sum(jnp.where(lane16 == 0, cnts[pl.ds(k * 16, 16)], 0))
        prefix = prefix + jnp.where(jnp.int32(k) < slot, ck, jnp.int32(0))
    astart = pl.multiple_of(prefix - jax.lax.rem(prefix, 16), 16)
    pad = prefix - astart

    def comp_body(g, off):
        valid = valid_at(g)
        for f in range(8):
            v = bufin[pl.ds(f * CH + g * 16, 16)]
            plsc.store_compressed(bufout.at[pl.ds(f * CW + off, 16)], v,
                                  mask=valid)
        lv = lbin[pl.ds(g * 16, 16)]
        plsc.store_compressed(lbout.at[pl.ds(off, 16)], lv, mask=valid)
        pc = plsc.all_reduce_population_count(valid)
        return off + jnp.sum(jnp.where(lane16 == 0, pc, 0))

    jax.lax.fori_loop(0, CH // 16, comp_body, pad)

    for r in range(NSLOT):
        plsc.subcore_barrier()

        @pl.when(slot == r)
        def _commit():
            for f in range(8):
                dst = pl.multiple_of((batch * 8 + f) * OUTW + astart, 16)
                pltpu.sync_copy(outf.at[pl.ds(dst, 16)], mrgf)
                head = bufout[pl.ds(f * CW, 16)]
                bufout[pl.ds(f * CW, 16)] = jnp.where(
                    lane16 < pad, mrgf[...], head)
                pltpu.sync_copy(bufout.at[pl.ds(f * CW, CW)],
                                outf.at[pl.ds(dst, CW)])
            ldst = pl.multiple_of(batch * OUTW + astart, 16)
            pltpu.sync_copy(outl.at[pl.ds(ldst, 16)], mrgl)
            lhead = lbout[pl.ds(0, 16)]
            lbout[pl.ds(0, 16)] = jnp.where(lane16 < pad, mrgl[...], lhead)
            pltpu.sync_copy(lbout, outl.at[pl.ds(ldst, CW)])


def _nms_body(boxc_ref, lbc_ref, rois_ref, scores_ref, labels_ref,
              x1_ref, x2_ref, y1_ref, y2_ref, ta_ref):
    B = boxc_ref.shape[0]
    flat = (jax.lax.broadcasted_iota(jnp.int32, (NR, C), 0) * C
            + jax.lax.broadcasted_iota(jnp.int32, (NR, C), 1))
    lane = jax.lax.broadcasted_iota(jnp.int32, (1, C), 1)

    for b in range(B):
        dx = boxc_ref[b, 3]
        dy = boxc_ref[b, 4]
        x1_ref[b] = boxc_ref[b, 0] - dx * 0.5
        x2_ref[b] = boxc_ref[b, 0] + dx * 0.5
        y1_ref[b] = boxc_ref[b, 1] - dy * 0.5
        y2_ref[b] = boxc_ref[b, 1] + dy * 0.5
        ta_ref[b] = (dx * dy) * NMS_THRESH

    rois_ref[...] = jnp.zeros_like(rois_ref)
    scores_ref[...] = jnp.zeros_like(scores_ref)
    labels_ref[...] = jnp.ones_like(labels_ref)

    def red_max(a):
        return jnp.max(jnp.max(a, axis=0, keepdims=True), axis=1,
                       keepdims=True)

    def red_min(a):
        return jnp.min(jnp.min(a, axis=0, keepdims=True), axis=1,
                       keepdims=True)

    def red_sum(a):
        return jnp.sum(jnp.sum(a, axis=0, keepdims=True), axis=1,
                       keepdims=True)

    def body(i, mss):
        out = []
        for b in range(B):
            ms = mss[b]
            m = red_max(ms)
            keep = m >= 0.0
            jv = red_min(jnp.where(ms == m, flat, jnp.int32(2 ** 30)))
            onehot = flat == jv

            rows = [jnp.sum(jnp.where(onehot, boxc_ref[b, f], 0.0),
                            axis=0, keepdims=True) for f in range(8)]
            v8 = jnp.sum(jnp.concatenate(rows, axis=0), axis=1,
                         keepdims=True)
            vals = [v8[f:f + 1, 0:1] for f in range(8)]
            lval = red_sum(jnp.where(onehot, lbc_ref[b], 0))

            xj, yj, dxj, dyj = vals[0], vals[1], vals[3], vals[4]
            x1j = xj - dxj * 0.5
            x2j = xj + dxj * 0.5
            y1j = yj - dyj * 0.5
            y2j = yj + dyj * 0.5
            saj = NMS_THRESH * (dxj * dyj + 1e-8)

            iw = jnp.maximum(
                jnp.minimum(x2j, x2_ref[b]) - jnp.maximum(x1j, x1_ref[b]),
                0.0)
            ih = jnp.maximum(
                jnp.minimum(y2j, y2_ref[b]) - jnp.maximum(y1j, y1_ref[b]),
                0.0)
            inter = iw * ih
            supp = (1.0 + NMS_THRESH) * inter > saj + ta_ref[b]
            out.append(jnp.where(keep & (supp | onehot), -1.0, ms))

            for f in range(7):
                rois_ref[b, pl.ds(i, 1), f:f + 1] = jnp.where(
                    keep, vals[f], 0.0)
            scores_ref[b, pl.ds(i, 1), 0:1] = jnp.where(keep, vals[7], 0.0)
            labels_ref[b, pl.ds(i, 1), 0:1] = jnp.where(
                keep, lval + 1, jnp.int32(1))
        return tuple(out)

    jax.lax.fori_loop(0, NMS_POST, body,
                      tuple(boxc_ref[b, 7] for b in range(B)))


@jax.jit
def kernel(batch_box_preds, batch_cls_preds):
    B, N, _ = batch_box_preds.shape
    boxes = jnp.moveaxis(batch_box_preds, 2, 1)
    boxes = jnp.pad(boxes, ((0, 0), (0, 0), (0, PN - N)))
    boxes = boxes.reshape(B, 7, R, C)
    cls = jnp.moveaxis(batch_cls_preds, 2, 1)
    cls = jnp.pad(cls, ((0, 0), (0, 0), (0, PN - N)), constant_values=-1.0)
    cls = cls.reshape(B, 3, R, C)

    scp, lbp, thr = pl.pallas_call(
        _thresh_body,
        in_specs=[
            pl.BlockSpec((B, 3, R, C), lambda: (0, 0, 0, 0)),
        ],
        out_specs=[
            pl.BlockSpec((B, R, C), lambda: (0, 0, 0)),
            pl.BlockSpec((B, R, C), lambda: (0, 0, 0)),
            pl.BlockSpec((B * NSLOT, C), lambda: (0, 0)),
        ],
        out_shape=[
            jax.ShapeDtypeStruct((B, R, C), jnp.float32),
            jax.ShapeDtypeStruct((B, R, C), jnp.int32),
            jax.ShapeDtypeStruct((B * NSLOT, C), jnp.int32),
        ],
    )(cls)

    compact = pl.kernel(
        _compact_body,
        out_type=[
            jax.ShapeDtypeStruct((B * 8 * OUTW,), jnp.float32),
            jax.ShapeDtypeStruct((B * OUTW,), jnp.int32),
        ],
        mesh=plsc.VectorSubcoreMesh(core_axis_name="c",
                                    subcore_axis_name="s",
                                    num_cores=2, num_subcores=16),
        compiler_params=pltpu.CompilerParams(needs_layout_passes=False),
        scratch_types=[
            pltpu.VMEM((8 * CH,), jnp.float32),
            pltpu.VMEM((CH,), jnp.int32),
            pltpu.VMEM((8 * CW,), jnp.float32),
            pltpu.VMEM((CW,), jnp.int32),
            pltpu.VMEM((16,), jnp.int32),
            pltpu.VMEM((16,), jnp.int32),
            pltpu.VMEM((NSLOT * 16,), jnp.int32),
            pltpu.VMEM((16,), jnp.float32),
            pltpu.VMEM((16,), jnp.int32),
            pltpu.VMEM_SHARED((16 * 16,), jnp.int32),
        ],
    )
    outf, outl = compact(boxes.reshape(B * 7 * PN), scp.reshape(B * PN),
                         lbp.reshape(B * PN), thr.reshape(B * NSLOT * C))

    boxc = outf.reshape(B, 8, OUTW // C, C)
    lbc = outl.reshape(B, OUTW // C, C)

    rois, scores, labels = pl.pallas_call(
        _nms_body,
        grid=(1,),
        in_specs=[
            pl.BlockSpec((B, 8, NR, C), lambda i: (0, 0, 0, 0)),
            pl.BlockSpec((B, NR, C), lambda i: (0, 0, 0)),
        ],
        out_specs=[
            pl.BlockSpec((B, NMS_POST, 7), lambda i: (0, 0, 0)),
            pl.BlockSpec((B, NMS_POST, 1), lambda i: (0, 0, 0)),
            pl.BlockSpec((B, NMS_POST, 1), lambda i: (0, 0, 0)),
        ],
        out_shape=[
            jax.ShapeDtypeStruct((B, NMS_POST, 7), jnp.float32),
            jax.ShapeDtypeStruct((B, NMS_POST, 1), jnp.float32),
            jax.ShapeDtypeStruct((B, NMS_POST, 1), jnp.int32),
        ],
        scratch_shapes=[pltpu.VMEM((B, NR, C), jnp.float32)] * 5,
    )(boxc, lbc)
    return rois, scores.reshape(B, NMS_POST), labels.reshape(B, NMS_POST)


# speedup vs baseline: 265.2682x; 1.1100x over previous
import functools

import jax
import jax.numpy as jnp
from jax.experimental import pallas as pl
from jax.experimental.pallas import tpu as pltpu
from jax.experimental.pallas import tpu_sc as plsc

NMS_PRE = 4096
NMS_POST = 512
NMS_THRESH = 0.8

R = 160
C = 128
PN = R * C
NSLOT = 8
CH = PN // NSLOT
CW = CH + 32
OUTW = 8192
NR = NMS_PRE // C


def _thresh_body(cls_ref, sc_ref, lb_ref, thr_ref):
    B = cls_ref.shape[0]
    flat = (jax.lax.broadcasted_iota(jnp.int32, (R, C), 0) * C
            + jax.lax.broadcasted_iota(jnp.int32, (R, C), 1))

    for b in range(B):
        c0 = cls_ref[b, 0]
        c1 = cls_ref[b, 1]
        c2 = cls_ref[b, 2]
        sc_ref[b] = jnp.maximum(jnp.maximum(c0, c1), c2)
        lb_ref[b] = jnp.where(c2 > jnp.maximum(c0, c1), 2,
                              jnp.where(c1 > c0, 1, 0)).astype(jnp.int32)

    def bs_val(_, carry):
        lo, hi = carry
        nlo, nhi = [], []
        for b in range(B):
            bits = jax.lax.bitcast_convert_type(sc_ref[b], jnp.int32)
            mid = lo[b] + jax.lax.div(hi[b] - lo[b], 2)
            cnt = jnp.sum(jnp.where(bits >= mid, 1, 0))
            cond = cnt >= NMS_PRE
            nlo.append(jnp.where(cond, mid, lo[b]))
            nhi.append(jnp.where(cond, hi[b], mid))
        return tuple(nlo), tuple(nhi)

    t_bits, _ = jax.lax.fori_loop(
        0, 30, bs_val,
        ((jnp.int32(0),) * B, (jnp.int32(0x40000000),) * B))

    n_ties = []
    for b in range(B):
        bits = jax.lax.bitcast_convert_type(sc_ref[b], jnp.int32)
        n_ties.append(NMS_PRE - jnp.sum(jnp.where(bits > t_bits[b], 1, 0)))

    def bs_idx(_, carry):
        lo, hi = carry
        nlo, nhi = [], []
        for b in range(B):
            bits = jax.lax.bitcast_convert_type(sc_ref[b], jnp.int32)
            mid = lo[b] + jax.lax.div(hi[b] - lo[b], 2)
            cnt = jnp.sum(
                jnp.where((bits == t_bits[b]) & (flat <= mid), 1, 0))
            cond = cnt >= n_ties[b]
            nlo.append(jnp.where(cond, lo[b], mid))
            nhi.append(jnp.where(cond, mid, hi[b]))
        return tuple(nlo), tuple(nhi)

    _, tie_idx = jax.lax.fori_loop(
        0, 16, bs_idx,
        ((jnp.int32(-1),) * B, (jnp.int32(PN - 1),) * B))

    thr_ref[...] = jnp.zeros_like(thr_ref)
    for b in range(B):
        thr_ref[NSLOT * b:NSLOT * b + 1, 0:1] = jnp.full(
            (1, 1), t_bits[b], jnp.int32)
        thr_ref[NSLOT * b:NSLOT * b + 1, 1:2] = jnp.full(
            (1, 1), tie_idx[b], jnp.int32)


def _compact_body(boxflat, scflat, lbflat, thr, outf, outl,
                  bufin, lbin, bufout, lbout, thrv, cntv, cnts,
                  mrgf, mrgl, shared, dsem):
    c = jax.lax.axis_index("c")
    s = jax.lax.axis_index("s")
    batch = c * 2 + jax.lax.div(s, NSLOT)
    slot = jax.lax.rem(s, NSLOT)
    start = slot * CH
    lane16 = jax.lax.broadcasted_iota(jnp.int32, (16,), 0)

    pltpu.sync_copy(
        thr.at[pl.ds(pl.multiple_of(batch * NSLOT * C, 16), 16)], thrv)
    tv = thrv[...]
    t_bits = jnp.sum(jnp.where(lane16 == 0, tv, 0))
    tie = jnp.sum(jnp.where(lane16 == 1, tv, 0))

    copies = []
    for f in range(7):
        copies.append(pltpu.async_copy(
            boxflat.at[pl.ds(
                pl.multiple_of((batch * 7 + f) * PN + start, 16), CH)],
            bufin.at[pl.ds(f * CH, CH)], dsem))
    copies.append(pltpu.async_copy(
        scflat.at[pl.ds(pl.multiple_of(batch * PN + start, 16), CH)],
        bufin.at[pl.ds(7 * CH, CH)], dsem))
    copies.append(pltpu.async_copy(
        lbflat.at[pl.ds(pl.multiple_of(batch * PN + start, 16), CH)],
        lbin, dsem))
    for cp in copies:
        cp.wait()

    def valid_at(g):
        sv = bufin[pl.ds(7 * CH + g * 16, 16)]
        bits = plsc.bitcast(sv, jnp.int32)
        gi = start + g * 16 + lane16
        return (bits > t_bits) | ((bits == t_bits) & (gi <= tie))

    def count_body(g, acc):
        return acc + jnp.where(valid_at(g), 1, 0)

    acc = jax.lax.fori_loop(0, CH // 16, count_body,
                            jnp.zeros((16,), jnp.int32))
    cnt = jnp.sum(acc)

    cntv[...] = jnp.full((16,), cnt, jnp.int32)
    pltpu.sync_copy(cntv, shared.at[pl.ds(pl.multiple_of(s * 16, 16), 16)])
    plsc.subcore_barrier()
    base = jax.lax.div(s, NSLOT) * NSLOT
    pltpu.sync_copy(
        shared.at[pl.ds(pl.multiple_of(base * 16, 16), NSLOT * 16)], cnts)
    prefix = jnp.int32(0)
    for k in range(NSLOT):
        ck = jnp.sum(jnp.where(lane16 == 0, cnts[pl.ds(k * 16, 16)], 0))
        prefix = prefix + jnp.where(jnp.int32(k) < slot, ck, jnp.int32(0))
    astart = pl.multiple_of(prefix - jax.lax.rem(prefix, 16), 16)
    pad = prefix - astart

    def comp_body(g, off):
        valid = valid_at(g)
        for f in range(8):
            v = bufin[pl.ds(f * CH + g * 16, 16)]
            plsc.store_compressed(bufout.at[pl.ds(f * CW + off, 16)], v,
                                  mask=valid)
        lv = lbin[pl.ds(g * 16, 16)]
        plsc.store_compressed(lbout.at[pl.ds(off, 16)], lv, mask=valid)
        pc = plsc.all_reduce_population_count(valid)
        return off + pc[0]

    jax.lax.fori_loop(0, CH // 16, comp_body, pad)

    for r in range(NSLOT):
        plsc.subcore_barrier()

        @pl.when(slot == r)
        def _commit():
            ldst = pl.multiple_of(batch * OUTW + astart, 16)
            rd = [pltpu.async_copy(
                outf.at[pl.ds(pl.multiple_of(
                    (batch * 8 + f) * OUTW + astart, 16), 16)],
                mrgf.at[pl.ds(f * 16, 16)], dsem) for f in range(8)]
            rd.append(pltpu.async_copy(outl.at[pl.ds(ldst, 16)], mrgl,
                                       dsem))
            for cp in rd:
                cp.wait()
            for f in range(8):
                head = bufout[pl.ds(f * CW, 16)]
                bufout[pl.ds(f * CW, 16)] = jnp.where(
                    lane16 < pad, mrgf[pl.ds(f * 16, 16)], head)
            lhead = lbout[pl.ds(0, 16)]
            lbout[pl.ds(0, 16)] = jnp.where(lane16 < pad, mrgl[...], lhead)
            wr = [pltpu.async_copy(
                bufout.at[pl.ds(f * CW, CW)],
                outf.at[pl.ds(pl.multiple_of(
                    (batch * 8 + f) * OUTW + astart, 16), CW)], dsem)
                for f in range(8)]
            wr.append(pltpu.async_copy(lbout, outl.at[pl.ds(ldst, CW)],
                                       dsem))
            for cp in wr:
                cp.wait()


def _nms_body(boxc_ref, lbc_ref, rois_ref, scores_ref, labels_ref,
              x1_ref, x2_ref, y1_ref, y2_ref, ta_ref):
    B = boxc_ref.shape[0]
    flat = (jax.lax.broadcasted_iota(jnp.int32, (NR, C), 0) * C
            + jax.lax.broadcasted_iota(jnp.int32, (NR, C), 1))
    lane = jax.lax.broadcasted_iota(jnp.int32, (1, C), 1)

    for b in range(B):
        dx = boxc_ref[b, 3]
        dy = boxc_ref[b, 4]
        x1_ref[b] = boxc_ref[b, 0] - dx * 0.5
        x2_ref[b] = boxc_ref[b, 0] + dx * 0.5
        y1_ref[b] = boxc_ref[b, 1] - dy * 0.5
        y2_ref[b] = boxc_ref[b, 1] + dy * 0.5
        ta_ref[b] = (dx * dy) * NMS_THRESH

    rois_ref[...] = jnp.zeros_like(rois_ref)
    scores_ref[...] = jnp.zeros_like(scores_ref)
    labels_ref[...] = jnp.ones_like(labels_ref)

    def red_max(a):
        return jnp.max(jnp.max(a, axis=0, keepdims=True), axis=1,
                       keepdims=True)

    def red_min(a):
        return jnp.min(jnp.min(a, axis=0, keepdims=True), axis=1,
                       keepdims=True)

    def red_sum(a):
        return jnp.sum(jnp.sum(a, axis=0, keepdims=True), axis=1,
                       keepdims=True)

    def body(i, mss):
        out = []
        for b in range(B):
            ms = mss[b]
            m = red_max(ms)
            keep = m >= 0.0
            jv = red_min(jnp.where(ms == m, flat, jnp.int32(2 ** 30)))
            onehot = flat == jv

            rows = [jnp.sum(jnp.where(onehot, boxc_ref[b, f], 0.0),
                            axis=0, keepdims=True) for f in range(8)]
            v8 = jnp.sum(jnp.concatenate(rows, axis=0), axis=1,
                         keepdims=True)
            vals = [v8[f:f + 1, 0:1] for f in range(8)]
            lval = red_sum(jnp.where(onehot, lbc_ref[b], 0))

            xj, yj, dxj, dyj = vals[0], vals[1], vals[3], vals[4]
            x1j = xj - dxj * 0.5
            x2j = xj + dxj * 0.5
            y1j = yj - dyj * 0.5
            y2j = yj + dyj * 0.5
            saj = NMS_THRESH * (dxj * dyj + 1e-8)

            iw = jnp.maximum(
                jnp.minimum(x2j, x2_ref[b]) - jnp.maximum(x1j, x1_ref[b]),
                0.0)
            ih = jnp.maximum(
                jnp.minimum(y2j, y2_ref[b]) - jnp.maximum(y1j, y1_ref[b]),
                0.0)
            inter = iw * ih
            supp = (1.0 + NMS_THRESH) * inter > saj + ta_ref[b]
            out.append(jnp.where(keep & (supp | onehot), -1.0, ms))

            for f in range(7):
                rois_ref[b, pl.ds(i, 1), f:f + 1] = jnp.where(
                    keep, vals[f], 0.0)
            scores_ref[b, pl.ds(i, 1), 0:1] = jnp.where(keep, vals[7], 0.0)
            labels_ref[b, pl.ds(i, 1), 0:1] = jnp.where(
                keep, lval + 1, jnp.int32(1))
        return tuple(out)

    jax.lax.fori_loop(0, NMS_POST, body,
                      tuple(boxc_ref[b, 7] for b in range(B)))


@jax.jit
def kernel(batch_box_preds, batch_cls_preds):
    B, N, _ = batch_box_preds.shape
    boxes = jnp.moveaxis(batch_box_preds, 2, 1)
    boxes = jnp.pad(boxes, ((0, 0), (0, 0), (0, PN - N)))
    boxes = boxes.reshape(B, 7, R, C)
    cls = jnp.moveaxis(batch_cls_preds, 2, 1)
    cls = jnp.pad(cls, ((0, 0), (0, 0), (0, PN - N)), constant_values=-1.0)
    cls = cls.reshape(B, 3, R, C)

    scp, lbp, thr = pl.pallas_call(
        _thresh_body,
        in_specs=[
            pl.BlockSpec((B, 3, R, C), lambda: (0, 0, 0, 0)),
        ],
        out_specs=[
            pl.BlockSpec((B, R, C), lambda: (0, 0, 0)),
            pl.BlockSpec((B, R, C), lambda: (0, 0, 0)),
            pl.BlockSpec((B * NSLOT, C), lambda: (0, 0)),
        ],
        out_shape=[
            jax.ShapeDtypeStruct((B, R, C), jnp.float32),
            jax.ShapeDtypeStruct((B, R, C), jnp.int32),
            jax.ShapeDtypeStruct((B * NSLOT, C), jnp.int32),
        ],
    )(cls)

    compact = pl.kernel(
        _compact_body,
        out_type=[
            jax.ShapeDtypeStruct((B * 8 * OUTW,), jnp.float32),
            jax.ShapeDtypeStruct((B * OUTW,), jnp.int32),
        ],
        mesh=plsc.VectorSubcoreMesh(core_axis_name="c",
                                    subcore_axis_name="s",
                                    num_cores=2, num_subcores=16),
        compiler_params=pltpu.CompilerParams(needs_layout_passes=False),
        scratch_types=[
            pltpu.VMEM((8 * CH,), jnp.float32),
            pltpu.VMEM((CH,), jnp.int32),
            pltpu.VMEM((8 * CW,), jnp.float32),
            pltpu.VMEM((CW,), jnp.int32),
            pltpu.VMEM((16,), jnp.int32),
            pltpu.VMEM((16,), jnp.int32),
            pltpu.VMEM((NSLOT * 16,), jnp.int32),
            pltpu.VMEM((8 * 16,), jnp.float32),
            pltpu.VMEM((16,), jnp.int32),
            pltpu.VMEM_SHARED((16 * 16,), jnp.int32),
            pltpu.SemaphoreType.DMA,
        ],
    )
    outf, outl = compact(boxes.reshape(B * 7 * PN), scp.reshape(B * PN),
                         lbp.reshape(B * PN), thr.reshape(B * NSLOT * C))

    boxc = outf.reshape(B, 8, OUTW // C, C)
    lbc = outl.reshape(B, OUTW // C, C)

    rois, scores, labels = pl.pallas_call(
        _nms_body,
        grid=(1,),
        in_specs=[
            pl.BlockSpec((B, 8, NR, C), lambda i: (0, 0, 0, 0)),
            pl.BlockSpec((B, NR, C), lambda i: (0, 0, 0)),
        ],
        out_specs=[
            pl.BlockSpec((B, NMS_POST, 7), lambda i: (0, 0, 0)),
            pl.BlockSpec((B, NMS_POST, 1), lambda i: (0, 0, 0)),
            pl.BlockSpec((B, NMS_POST, 1), lambda i: (0, 0, 0)),
        ],
        out_shape=[
            jax.ShapeDtypeStruct((B, NMS_POST, 7), jnp.float32),
            jax.ShapeDtypeStruct((B, NMS_POST, 1), jnp.float32),
            jax.ShapeDtypeStruct((B, NMS_POST, 1), jnp.int32),
        ],
        scratch_shapes=[pltpu.VMEM((B, NR, C), jnp.float32)] * 5,
    )(boxc, lbc)
    return rois, scores.reshape(B, NMS_POST), labels.reshape(B, NMS_POST)


# speedup vs baseline: 271.4661x; 1.0234x over previous
import functools

import jax
import jax.numpy as jnp
from jax.experimental import pallas as pl
from jax.experimental.pallas import tpu as pltpu
from jax.experimental.pallas import tpu_sc as plsc

NMS_PRE = 4096
NMS_POST = 512
NMS_THRESH = 0.8

R = 160
C = 128
PN = R * C
NSLOT = 8
CH = PN // NSLOT
CW = CH + 32
OUTW = 8192
NR = NMS_PRE // C


def _thresh_body(cls_ref, sc_ref, lb_ref, thr_ref):
    B = cls_ref.shape[0]
    flat = (jax.lax.broadcasted_iota(jnp.int32, (R, C), 0) * C
            + jax.lax.broadcasted_iota(jnp.int32, (R, C), 1))

    for b in range(B):
        c0 = cls_ref[b, 0]
        c1 = cls_ref[b, 1]
        c2 = cls_ref[b, 2]
        sc_ref[b] = jnp.maximum(jnp.maximum(c0, c1), c2)
        lb_ref[b] = jnp.where(c2 > jnp.maximum(c0, c1), 2,
                              jnp.where(c1 > c0, 1, 0)).astype(jnp.int32)

    def bs_val(_, carry):
        lo, hi = carry
        nlo, nhi = [], []
        for b in range(B):
            bits = jax.lax.bitcast_convert_type(sc_ref[b], jnp.int32)
            mid = lo[b] + jax.lax.div(hi[b] - lo[b], 2)
            cnt = jnp.sum(jnp.where(bits >= mid, 1, 0))
            cond = cnt >= NMS_PRE
            nlo.append(jnp.where(cond, mid, lo[b]))
            nhi.append(jnp.where(cond, hi[b], mid))
        return tuple(nlo), tuple(nhi)

    t_bits, _ = jax.lax.fori_loop(
        0, 30, bs_val,
        ((jnp.int32(0),) * B, (jnp.int32(0x40000000),) * B))

    n_ties = []
    for b in range(B):
        bits = jax.lax.bitcast_convert_type(sc_ref[b], jnp.int32)
        n_ties.append(NMS_PRE - jnp.sum(jnp.where(bits > t_bits[b], 1, 0)))

    def bs_idx(_, carry):
        lo, hi = carry
        nlo, nhi = [], []
        for b in range(B):
            bits = jax.lax.bitcast_convert_type(sc_ref[b], jnp.int32)
            mid = lo[b] + jax.lax.div(hi[b] - lo[b], 2)
            cnt = jnp.sum(
                jnp.where((bits == t_bits[b]) & (flat <= mid), 1, 0))
            cond = cnt >= n_ties[b]
            nlo.append(jnp.where(cond, lo[b], mid))
            nhi.append(jnp.where(cond, mid, hi[b]))
        return tuple(nlo), tuple(nhi)

    _, tie_idx = jax.lax.fori_loop(
        0, 16, bs_idx,
        ((jnp.int32(-1),) * B, (jnp.int32(PN - 1),) * B))

    thr_ref[...] = jnp.zeros_like(thr_ref)
    for b in range(B):
        thr_ref[NSLOT * b:NSLOT * b + 1, 0:1] = jnp.full(
            (1, 1), t_bits[b], jnp.int32)
        thr_ref[NSLOT * b:NSLOT * b + 1, 1:2] = jnp.full(
            (1, 1), tie_idx[b], jnp.int32)


def _compact_body(boxflat, scflat, lbflat, thr, outf, outl,
                  bufin, lbin, bufout, lbout, thrv, cntv, cnts,
                  mrgf, mrgl, shared, dsem):
    c = jax.lax.axis_index("c")
    s = jax.lax.axis_index("s")
    batch = c * 2 + jax.lax.div(s, NSLOT)
    slot = jax.lax.rem(s, NSLOT)
    start = slot * CH
    lane16 = jax.lax.broadcasted_iota(jnp.int32, (16,), 0)

    pltpu.sync_copy(
        thr.at[pl.ds(pl.multiple_of(batch * NSLOT * C, 16), 16)], thrv)
    tv = thrv[...]
    t_bits = jnp.sum(jnp.where(lane16 == 0, tv, 0))
    tie = jnp.sum(jnp.where(lane16 == 1, tv, 0))

    copies = []
    for f in range(7):
        copies.append(pltpu.async_copy(
            boxflat.at[pl.ds(
                pl.multiple_of((batch * 7 + f) * PN + start, 16), CH)],
            bufin.at[pl.ds(f * CH, CH)], dsem))
    copies.append(pltpu.async_copy(
        scflat.at[pl.ds(pl.multiple_of(batch * PN + start, 16), CH)],
        bufin.at[pl.ds(7 * CH, CH)], dsem))
    copies.append(pltpu.async_copy(
        lbflat.at[pl.ds(pl.multiple_of(batch * PN + start, 16), CH)],
        lbin, dsem))
    for cp in copies:
        cp.wait()

    def valid_at(g):
        sv = bufin[pl.ds(7 * CH + g * 16, 16)]
        bits = plsc.bitcast(sv, jnp.int32)
        gi = start + g * 16 + lane16
        return (bits > t_bits) | ((bits == t_bits) & (gi <= tie))

    def count_body(g, acc):
        return acc + jnp.where(valid_at(g), 1, 0)

    acc = jax.lax.fori_loop(0, CH // 16, count_body,
                            jnp.zeros((16,), jnp.int32))
    cnt = jnp.sum(acc)

    cntv[...] = jnp.full((16,), cnt, jnp.int32)
    pltpu.sync_copy(cntv, shared.at[pl.ds(pl.multiple_of(s * 16, 16), 16)])
    plsc.subcore_barrier()
    base = jax.lax.div(s, NSLOT) * NSLOT
    pltpu.sync_copy(
        shared.at[pl.ds(pl.multiple_of(base * 16, 16), NSLOT * 16)], cnts)
    prefix = jnp.int32(0)
    for k in range(NSLOT):
        ck = jnp.sum(jnp.where(lane16 == 0, cnts[pl.ds(k * 16, 16)], 0))
        prefix = prefix + jnp.where(jnp.int32(k) < slot, ck, jnp.int32(0))
    astart = pl.multiple_of(prefix - jax.lax.rem(prefix, 16), 16)
    pad = prefix - astart

    def comp_body(g, off):
        valid = valid_at(g)
        for f in range(8):
            v = bufin[pl.ds(f * CH + g * 16, 16)]
            plsc.store_compressed(bufout.at[pl.ds(f * CW + off, 16)], v,
                                  mask=valid)
        lv = lbin[pl.ds(g * 16, 16)]
        plsc.store_compressed(lbout.at[pl.ds(off, 16)], lv, mask=valid)
        pc = plsc.all_reduce_population_count(valid)
        return off + pc[0]

    jax.lax.fori_loop(0, CH // 16, comp_body, pad)

    for r in range(NSLOT):
        plsc.subcore_barrier()

        @pl.when(slot == r)
        def _commit():
            ldst = pl.multiple_of(batch * OUTW + astart, 16)
            rd = [pltpu.async_copy(
                outf.at[pl.ds(pl.multiple_of(
                    (batch * 8 + f) * OUTW + astart, 16), 16)],
                mrgf.at[pl.ds(f * 16, 16)], dsem) for f in range(8)]
            rd.append(pltpu.async_copy(outl.at[pl.ds(ldst, 16)], mrgl,
                                       dsem))
            for cp in rd:
                cp.wait()
            for f in range(8):
                head = bufout[pl.ds(f * CW, 16)]
                bufout[pl.ds(f * CW, 16)] = jnp.where(
                    lane16 < pad, mrgf[pl.ds(f * 16, 16)], head)
            lhead = lbout[pl.ds(0, 16)]
            lbout[pl.ds(0, 16)] = jnp.where(lane16 < pad, mrgl[...], lhead)
            wr = [pltpu.async_copy(
                bufout.at[pl.ds(f * CW, CW)],
                outf.at[pl.ds(pl.multiple_of(
                    (batch * 8 + f) * OUTW + astart, 16), CW)], dsem)
                for f in range(8)]
            wr.append(pltpu.async_copy(lbout, outl.at[pl.ds(ldst, CW)],
                                       dsem))
            for cp in wr:
                cp.wait()


def _nms_body(boxc_ref, lbc_ref, rois_ref, scores_ref, labels_ref,
              x1_ref, x2_ref, y1_ref, y2_ref, ta_ref):
    B = boxc_ref.shape[0]
    flat = (jax.lax.broadcasted_iota(jnp.int32, (NR, C), 0) * C
            + jax.lax.broadcasted_iota(jnp.int32, (NR, C), 1))
    lane = jax.lax.broadcasted_iota(jnp.int32, (1, C), 1)

    for b in range(B):
        dx = boxc_ref[b, 3]
        dy = boxc_ref[b, 4]
        x1_ref[b] = boxc_ref[b, 0] - dx * 0.5
        x2_ref[b] = boxc_ref[b, 0] + dx * 0.5
        y1_ref[b] = boxc_ref[b, 1] - dy * 0.5
        y2_ref[b] = boxc_ref[b, 1] + dy * 0.5
        ta_ref[b] = (dx * dy) * NMS_THRESH

    rois_ref[...] = jnp.zeros_like(rois_ref)
    scores_ref[...] = jnp.zeros_like(scores_ref)
    labels_ref[...] = jnp.ones_like(labels_ref)

    def red_max(a):
        return jnp.max(jnp.max(a, axis=0, keepdims=True), axis=1,
                       keepdims=True)

    def red_min(a):
        return jnp.min(jnp.min(a, axis=0, keepdims=True), axis=1,
                       keepdims=True)

    def red_sum(a):
        return jnp.sum(jnp.sum(a, axis=0, keepdims=True), axis=1,
                       keepdims=True)

    def step(i, mss):
        out = []
        for b in range(B):
            ms = mss[b]
            m = red_max(ms)
            keep = m >= 0.0
            jv = red_min(jnp.where(ms == m, flat, jnp.int32(2 ** 30)))
            onehot = flat == jv

            rows = [jnp.sum(jnp.where(onehot, boxc_ref[b, f], 0.0),
                            axis=0, keepdims=True) for f in range(8)]
            v8 = jnp.sum(jnp.concatenate(rows, axis=0), axis=1,
                         keepdims=True)
            vals = [v8[f:f + 1, 0:1] for f in range(8)]
            lval = red_sum(jnp.where(onehot, lbc_ref[b], 0))

            xj, yj, dxj, dyj = vals[0], vals[1], vals[3], vals[4]
            x1j = xj - dxj * 0.5
            x2j = xj + dxj * 0.5
            y1j = yj - dyj * 0.5
            y2j = yj + dyj * 0.5
            saj = NMS_THRESH * (dxj * dyj + 1e-8)

            iw = jnp.maximum(
                jnp.minimum(x2j, x2_ref[b]) - jnp.maximum(x1j, x1_ref[b]),
                0.0)
            ih = jnp.maximum(
                jnp.minimum(y2j, y2_ref[b]) - jnp.maximum(y1j, y1_ref[b]),
                0.0)
            inter = iw * ih
            supp = (1.0 + NMS_THRESH) * inter > saj + ta_ref[b]
            out.append(jnp.where(keep & (supp | onehot), -1.0, ms))

            for f in range(7):
                rois_ref[b, pl.ds(i, 1), f:f + 1] = jnp.where(
                    keep, vals[f], 0.0)
            scores_ref[b, pl.ds(i, 1), 0:1] = jnp.where(keep, vals[7], 0.0)
            labels_ref[b, pl.ds(i, 1), 0:1] = jnp.where(
                keep, lval + 1, jnp.int32(1))
        return tuple(out)

    def body(h, mss):
        return step(h * 2 + 1, step(h * 2, mss))

    jax.lax.fori_loop(0, NMS_POST // 2, body,
                      tuple(boxc_ref[b, 7] for b in range(B)))


@jax.jit
def kernel(batch_box_preds, batch_cls_preds):
    B, N, _ = batch_box_preds.shape
    boxes = jnp.moveaxis(batch_box_preds, 2, 1)
    boxes = jnp.pad(boxes, ((0, 0), (0, 0), (0, PN - N)))
    boxes = boxes.reshape(B, 7, R, C)
    cls = jnp.moveaxis(batch_cls_preds, 2, 1)
    cls = jnp.pad(cls, ((0, 0), (0, 0), (0, PN - N)), constant_values=-1.0)
    cls = cls.reshape(B, 3, R, C)

    scp, lbp, thr = pl.pallas_call(
        _thresh_body,
        in_specs=[
            pl.BlockSpec((B, 3, R, C), lambda: (0, 0, 0, 0)),
        ],
        out_specs=[
            pl.BlockSpec((B, R, C), lambda: (0, 0, 0)),
            pl.BlockSpec((B, R, C), lambda: (0, 0, 0)),
            pl.BlockSpec((B * NSLOT, C), lambda: (0, 0)),
        ],
        out_shape=[
            jax.ShapeDtypeStruct((B, R, C), jnp.float32),
            jax.ShapeDtypeStruct((B, R, C), jnp.int32),
            jax.ShapeDtypeStruct((B * NSLOT, C), jnp.int32),
        ],
    )(cls)

    compact = pl.kernel(
        _compact_body,
        out_type=[
            jax.ShapeDtypeStruct((B * 8 * OUTW,), jnp.float32),
            jax.ShapeDtypeStruct((B * OUTW,), jnp.int32),
        ],
        mesh=plsc.VectorSubcoreMesh(core_axis_name="c",
                                    subcore_axis_name="s",
                                    num_cores=2, num_subcores=16),
        compiler_params=pltpu.CompilerParams(needs_layout_passes=False),
        scratch_types=[
            pltpu.VMEM((8 * CH,), jnp.float32),
            pltpu.VMEM((CH,), jnp.int32),
            pltpu.VMEM((8 * CW,), jnp.float32),
            pltpu.VMEM((CW,), jnp.int32),
            pltpu.VMEM((16,), jnp.int32),
            pltpu.VMEM((16,), jnp.int32),
            pltpu.VMEM((NSLOT * 16,), jnp.int32),
            pltpu.VMEM((8 * 16,), jnp.float32),
            pltpu.VMEM((16,), jnp.int32),
            pltpu.VMEM_SHARED((16 * 16,), jnp.int32),
            pltpu.SemaphoreType.DMA,
        ],
    )
    outf, outl = compact(boxes.reshape(B * 7 * PN), scp.reshape(B * PN),
                         lbp.reshape(B * PN), thr.reshape(B * NSLOT * C))

    boxc = outf.reshape(B, 8, OUTW // C, C)
    lbc = outl.reshape(B, OUTW // C, C)

    rois, scores, labels = pl.pallas_call(
        _nms_body,
        grid=(1,),
        in_specs=[
            pl.BlockSpec((B, 8, NR, C), lambda i: (0, 0, 0, 0)),
            pl.BlockSpec((B, NR, C), lambda i: (0, 0, 0)),
        ],
        out_specs=[
            pl.BlockSpec((B, NMS_POST, 7), lambda i: (0, 0, 0)),
            pl.BlockSpec((B, NMS_POST, 1), lambda i: (0, 0, 0)),
            pl.BlockSpec((B, NMS_POST, 1), lambda i: (0, 0, 0)),
        ],
        out_shape=[
            jax.ShapeDtypeStruct((B, NMS_POST, 7), jnp.float32),
            jax.ShapeDtypeStruct((B, NMS_POST, 1), jnp.float32),
            jax.ShapeDtypeStruct((B, NMS_POST, 1), jnp.int32),
        ],
        scratch_shapes=[pltpu.VMEM((B, NR, C), jnp.float32)] * 5,
    )(boxc, lbc)
    return rois, scores.reshape(B, NMS_POST), labels.reshape(B, NMS_POST)


# speedup vs baseline: 279.1490x; 1.0283x over previous
import functools

import jax
import jax.numpy as jnp
from jax.experimental import pallas as pl
from jax.experimental.pallas import tpu as pltpu
from jax.experimental.pallas import tpu_sc as plsc

NMS_PRE = 4096
NMS_POST = 512
NMS_THRESH = 0.8

R = 160
C = 128
PN = R * C
NSLOT = 8
CH = PN // NSLOT
CW = CH + 32
OUTW = 8192
NR = NMS_PRE // C


def _thresh_body(cls_ref, sc_ref, lb_ref, thr_ref):
    B = cls_ref.shape[0]
    flat = (jax.lax.broadcasted_iota(jnp.int32, (R, C), 0) * C
            + jax.lax.broadcasted_iota(jnp.int32, (R, C), 1))

    for b in range(B):
        c0 = cls_ref[b, 0]
        c1 = cls_ref[b, 1]
        c2 = cls_ref[b, 2]
        sc_ref[b] = jnp.maximum(jnp.maximum(c0, c1), c2)
        lb_ref[b] = jnp.where(c2 > jnp.maximum(c0, c1), 2,
                              jnp.where(c1 > c0, 1, 0)).astype(jnp.int32)

    def bs_val(_, carry):
        lo, hi = carry
        nlo, nhi = [], []
        for b in range(B):
            bits = jax.lax.bitcast_convert_type(sc_ref[b], jnp.int32)
            mid = lo[b] + jax.lax.div(hi[b] - lo[b], 2)
            cnt = jnp.sum(jnp.where(bits >= mid, 1, 0))
            cond = cnt >= NMS_PRE
            nlo.append(jnp.where(cond, mid, lo[b]))
            nhi.append(jnp.where(cond, hi[b], mid))
        return tuple(nlo), tuple(nhi)

    t_bits, _ = jax.lax.fori_loop(
        0, 30, bs_val,
        ((jnp.int32(0),) * B, (jnp.int32(0x40000000),) * B))

    n_ties = []
    for b in range(B):
        bits = jax.lax.bitcast_convert_type(sc_ref[b], jnp.int32)
        n_ties.append(NMS_PRE - jnp.sum(jnp.where(bits > t_bits[b], 1, 0)))

    def bs_idx(_, carry):
        lo, hi = carry
        nlo, nhi = [], []
        for b in range(B):
            bits = jax.lax.bitcast_convert_type(sc_ref[b], jnp.int32)
            mid = lo[b] + jax.lax.div(hi[b] - lo[b], 2)
            cnt = jnp.sum(
                jnp.where((bits == t_bits[b]) & (flat <= mid), 1, 0))
            cond = cnt >= n_ties[b]
            nlo.append(jnp.where(cond, lo[b], mid))
            nhi.append(jnp.where(cond, mid, hi[b]))
        return tuple(nlo), tuple(nhi)

    _, tie_idx = jax.lax.fori_loop(
        0, 16, bs_idx,
        ((jnp.int32(-1),) * B, (jnp.int32(PN - 1),) * B))

    thr_ref[...] = jnp.zeros_like(thr_ref)
    for b in range(B):
        thr_ref[NSLOT * b:NSLOT * b + 1, 0:1] = jnp.full(
            (1, 1), t_bits[b], jnp.int32)
        thr_ref[NSLOT * b:NSLOT * b + 1, 1:2] = jnp.full(
            (1, 1), tie_idx[b], jnp.int32)


def _compact_body(boxflat, scflat, lbflat, thr, outf, outl,
                  bufin, lbin, bufout, lbout, thrv, cntv, cnts,
                  mrgf, mrgl, shared, dsem):
    c = jax.lax.axis_index("c")
    s = jax.lax.axis_index("s")
    batch = c * 2 + jax.lax.div(s, NSLOT)
    slot = jax.lax.rem(s, NSLOT)
    start = slot * CH
    lane16 = jax.lax.broadcasted_iota(jnp.int32, (16,), 0)

    pltpu.sync_copy(
        thr.at[pl.ds(pl.multiple_of(batch * NSLOT * C, 16), 16)], thrv)
    tv = thrv[...]
    t_bits = jnp.sum(jnp.where(lane16 == 0, tv, 0))
    tie = jnp.sum(jnp.where(lane16 == 1, tv, 0))

    copies = []
    for f in range(7):
        copies.append(pltpu.async_copy(
            boxflat.at[pl.ds(
                pl.multiple_of((batch * 7 + f) * PN + start, 16), CH)],
            bufin.at[pl.ds(f * CH, CH)], dsem))
    copies.append(pltpu.async_copy(
        scflat.at[pl.ds(pl.multiple_of(batch * PN + start, 16), CH)],
        bufin.at[pl.ds(7 * CH, CH)], dsem))
    copies.append(pltpu.async_copy(
        lbflat.at[pl.ds(pl.multiple_of(batch * PN + start, 16), CH)],
        lbin, dsem))
    for cp in copies:
        cp.wait()

    def valid_at(g):
        sv = bufin[pl.ds(7 * CH + g * 16, 16)]
        bits = plsc.bitcast(sv, jnp.int32)
        gi = start + g * 16 + lane16
        return (bits > t_bits) | ((bits == t_bits) & (gi <= tie))

    def count_body(g, acc):
        return acc + jnp.where(valid_at(g), 1, 0)

    acc = jax.lax.fori_loop(0, CH // 16, count_body,
                            jnp.zeros((16,), jnp.int32))
    cnt = jnp.sum(acc)

    cntv[...] = jnp.full((16,), cnt, jnp.int32)
    pltpu.sync_copy(cntv, shared.at[pl.ds(pl.multiple_of(s * 16, 16), 16)])
    plsc.subcore_barrier()
    base = jax.lax.div(s, NSLOT) * NSLOT
    pltpu.sync_copy(
        shared.at[pl.ds(pl.multiple_of(base * 16, 16), NSLOT * 16)], cnts)
    prefix = jnp.int32(0)
    for k in range(NSLOT):
        ck = jnp.sum(jnp.where(lane16 == 0, cnts[pl.ds(k * 16, 16)], 0))
        prefix = prefix + jnp.where(jnp.int32(k) < slot, ck, jnp.int32(0))
    astart = pl.multiple_of(prefix - jax.lax.rem(prefix, 16), 16)
    pad = prefix - astart

    def comp_body(g, off):
        valid = valid_at(g)
        for f in range(8):
            v = bufin[pl.ds(f * CH + g * 16, 16)]
            plsc.store_compressed(bufout.at[pl.ds(f * CW + off, 16)], v,
                                  mask=valid)
        lv = lbin[pl.ds(g * 16, 16)]
        plsc.store_compressed(lbout.at[pl.ds(off, 16)], lv, mask=valid)
        pc = plsc.all_reduce_population_count(valid)
        return off + pc[0]

    jax.lax.fori_loop(0, CH // 16, comp_body, pad)

    for r in range(NSLOT):
        plsc.subcore_barrier()

        @pl.when(slot == r)
        def _commit():
            ldst = pl.multiple_of(batch * OUTW + astart, 16)
            rd = [pltpu.async_copy(
                outf.at[pl.ds(pl.multiple_of(
                    (batch * 8 + f) * OUTW + astart, 16), 16)],
                mrgf.at[pl.ds(f * 16, 16)], dsem) for f in range(8)]
            rd.append(pltpu.async_copy(outl.at[pl.ds(ldst, 16)], mrgl,
                                       dsem))
            for cp in rd:
                cp.wait()
            for f in range(8):
                head = bufout[pl.ds(f * CW, 16)]
                bufout[pl.ds(f * CW, 16)] = jnp.where(
                    lane16 < pad, mrgf[pl.ds(f * 16, 16)], head)
            lhead = lbout[pl.ds(0, 16)]
            lbout[pl.ds(0, 16)] = jnp.where(lane16 < pad, mrgl[...], lhead)
            wr = [pltpu.async_copy(
                bufout.at[pl.ds(f * CW, CW)],
                outf.at[pl.ds(pl.multiple_of(
                    (batch * 8 + f) * OUTW + astart, 16), CW)], dsem)
                for f in range(8)]
            wr.append(pltpu.async_copy(lbout, outl.at[pl.ds(ldst, CW)],
                                       dsem))
            for cp in wr:
                cp.wait()


def _nms_body(boxc_ref, lbc_ref, rois_ref, scores_ref, labels_ref,
              x1_ref, x2_ref, y1_ref, y2_ref, ta_ref):
    B = boxc_ref.shape[0]
    flat = (jax.lax.broadcasted_iota(jnp.int32, (NR, C), 0) * C
            + jax.lax.broadcasted_iota(jnp.int32, (NR, C), 1))
    lane = jax.lax.broadcasted_iota(jnp.int32, (1, C), 1)

    for b in range(B):
        dx = boxc_ref[b, 3]
        dy = boxc_ref[b, 4]
        x1_ref[b] = boxc_ref[b, 0] - dx * 0.5
        x2_ref[b] = boxc_ref[b, 0] + dx * 0.5
        y1_ref[b] = boxc_ref[b, 1] - dy * 0.5
        y2_ref[b] = boxc_ref[b, 1] + dy * 0.5
        ta_ref[b] = (dx * dy) * NMS_THRESH

    rois_ref[...] = jnp.zeros_like(rois_ref)
    scores_ref[...] = jnp.zeros_like(scores_ref)
    labels_ref[...] = jnp.ones_like(labels_ref)

    def red_max(a):
        return jnp.max(jnp.max(a, axis=0, keepdims=True), axis=1,
                       keepdims=True)

    def red_min(a):
        return jnp.min(jnp.min(a, axis=0, keepdims=True), axis=1,
                       keepdims=True)

    def red_sum(a):
        return jnp.sum(jnp.sum(a, axis=0, keepdims=True), axis=1,
                       keepdims=True)

    def step(i, mss):
        out = []
        for b in range(B):
            ms = mss[b]
            m = red_max(ms)
            keep = m >= 0.0
            jv = red_min(jnp.where(ms == m, flat, jnp.int32(2 ** 30)))
            onehot = flat == jv

            rows = [jnp.sum(jnp.where(onehot, boxc_ref[b, f], 0.0),
                            axis=0, keepdims=True) for f in range(8)]
            v8 = jnp.sum(jnp.concatenate(rows, axis=0), axis=1,
                         keepdims=True)
            vals = [v8[f:f + 1, 0:1] for f in range(8)]
            lval = red_sum(jnp.where(onehot, lbc_ref[b], 0))

            xj, yj, dxj, dyj = vals[0], vals[1], vals[3], vals[4]
            x1j = xj - dxj * 0.5
            x2j = xj + dxj * 0.5
            y1j = yj - dyj * 0.5
            y2j = yj + dyj * 0.5
            saj = NMS_THRESH * (dxj * dyj + 1e-8)

            iw = jnp.maximum(
                jnp.minimum(x2j, x2_ref[b]) - jnp.maximum(x1j, x1_ref[b]),
                0.0)
            ih = jnp.maximum(
                jnp.minimum(y2j, y2_ref[b]) - jnp.maximum(y1j, y1_ref[b]),
                0.0)
            inter = iw * ih
            supp = (1.0 + NMS_THRESH) * inter > saj + ta_ref[b]
            out.append(jnp.where(keep & (supp | onehot), -1.0, ms))

            for f in range(7):
                rois_ref[b, pl.ds(i, 1), f:f + 1] = jnp.where(
                    keep, vals[f], 0.0)
            scores_ref[b, pl.ds(i, 1), 0:1] = jnp.where(keep, vals[7], 0.0)
            labels_ref[b, pl.ds(i, 1), 0:1] = jnp.where(
                keep, lval + 1, jnp.int32(1))
        return tuple(out)

    def body(h, mss):
        for u in range(4):
            mss = step(h * 4 + u, mss)
        return mss

    jax.lax.fori_loop(0, NMS_POST // 4, body,
                      tuple(boxc_ref[b, 7] for b in range(B)))


@jax.jit
def kernel(batch_box_preds, batch_cls_preds):
    B, N, _ = batch_box_preds.shape
    boxes = jnp.moveaxis(batch_box_preds, 2, 1)
    boxes = jnp.pad(boxes, ((0, 0), (0, 0), (0, PN - N)))
    boxes = boxes.reshape(B, 7, R, C)
    cls = jnp.moveaxis(batch_cls_preds, 2, 1)
    cls = jnp.pad(cls, ((0, 0), (0, 0), (0, PN - N)), constant_values=-1.0)
    cls = cls.reshape(B, 3, R, C)

    scp, lbp, thr = pl.pallas_call(
        _thresh_body,
        in_specs=[
            pl.BlockSpec((B, 3, R, C), lambda: (0, 0, 0, 0)),
        ],
        out_specs=[
            pl.BlockSpec((B, R, C), lambda: (0, 0, 0)),
            pl.BlockSpec((B, R, C), lambda: (0, 0, 0)),
            pl.BlockSpec((B * NSLOT, C), lambda: (0, 0)),
        ],
        out_shape=[
            jax.ShapeDtypeStruct((B, R, C), jnp.float32),
            jax.ShapeDtypeStruct((B, R, C), jnp.int32),
            jax.ShapeDtypeStruct((B * NSLOT, C), jnp.int32),
        ],
    )(cls)

    compact = pl.kernel(
        _compact_body,
        out_type=[
            jax.ShapeDtypeStruct((B * 8 * OUTW,), jnp.float32),
            jax.ShapeDtypeStruct((B * OUTW,), jnp.int32),
        ],
        mesh=plsc.VectorSubcoreMesh(core_axis_name="c",
                                    subcore_axis_name="s",
                                    num_cores=2, num_subcores=16),
        compiler_params=pltpu.CompilerParams(needs_layout_passes=False),
        scratch_types=[
            pltpu.VMEM((8 * CH,), jnp.float32),
            pltpu.VMEM((CH,), jnp.int32),
            pltpu.VMEM((8 * CW,), jnp.float32),
            pltpu.VMEM((CW,), jnp.int32),
            pltpu.VMEM((16,), jnp.int32),
            pltpu.VMEM((16,), jnp.int32),
            pltpu.VMEM((NSLOT * 16,), jnp.int32),
            pltpu.VMEM((8 * 16,), jnp.float32),
            pltpu.VMEM((16,), jnp.int32),
            pltpu.VMEM_SHARED((16 * 16,), jnp.int32),
            pltpu.SemaphoreType.DMA,
        ],
    )
    outf, outl = compact(boxes.reshape(B * 7 * PN), scp.reshape(B * PN),
                         lbp.reshape(B * PN), thr.reshape(B * NSLOT * C))

    boxc = outf.reshape(B, 8, OUTW // C, C)
    lbc = outl.reshape(B, OUTW // C, C)

    rois, scores, labels = pl.pallas_call(
        _nms_body,
        grid=(1,),
        in_specs=[
            pl.BlockSpec((B, 8, NR, C), lambda i: (0, 0, 0, 0)),
            pl.BlockSpec((B, NR, C), lambda i: (0, 0, 0)),
        ],
        out_specs=[
            pl.BlockSpec((B, NMS_POST, 7), lambda i: (0, 0, 0)),
            pl.BlockSpec((B, NMS_POST, 1), lambda i: (0, 0, 0)),
            pl.BlockSpec((B, NMS_POST, 1), lambda i: (0, 0, 0)),
        ],
        out_shape=[
            jax.ShapeDtypeStruct((B, NMS_POST, 7), jnp.float32),
            jax.ShapeDtypeStruct((B, NMS_POST, 1), jnp.float32),
            jax.ShapeDtypeStruct((B, NMS_POST, 1), jnp.int32),
        ],
        scratch_shapes=[pltpu.VMEM((B, NR, C), jnp.float32)] * 5,
    )(boxc, lbc)
    return rois, scores.reshape(B, NMS_POST), labels.reshape(B, NMS_POST)
